# baseline (speedup 1.0000x reference)
import jax
import jax.numpy as jnp
from jax import lax
from jax.experimental import pallas as pl
from jax.experimental.pallas import tpu as pltpu

N = 2048
HID = 256
OUT = 128
F32 = jnp.float32
BF16 = jnp.bfloat16

_TN = (((0,), (0,)), ((), ()))
_NN = (((1,), (0,)), ((), ()))
_PARAMS = pltpu.CompilerParams(
    dimension_semantics=("parallel", "parallel"))


def _dinv(adj):
    bj = 512

    def body(adj_ref, out_ref):
        j = pl.program_id(0)
        t = adj_ref[...]
        b = jnp.where(t != 0, 1.0, 0.0)
        rid = lax.broadcasted_iota(jnp.int32, t.shape, 0)
        cid = j * bj + lax.broadcasted_iota(jnp.int32, t.shape, 1)
        d = jnp.where(rid == cid, 1.0, b)
        deg = jnp.sum(d, axis=0)
        out_ref[...] = 1.0 / jnp.sqrt(deg)

    return pl.pallas_call(
        body,
        grid=(N // bj,),
        in_specs=[pl.BlockSpec((N, bj), lambda j: (0, j))],
        out_specs=pl.BlockSpec((bj,), lambda j: (j,)),
        out_shape=jax.ShapeDtypeStruct((N,), F32),
    )(adj)


def _norm(adj, dinv_c, dinv_r):
    bj = 512

    def body(adj_ref, dc_ref, dr_ref, out_ref):
        j = pl.program_id(0)
        t = adj_ref[...]
        b = jnp.where(t != 0, 1.0, 0.0)
        rid = lax.broadcasted_iota(jnp.int32, t.shape, 0)
        cid = j * bj + lax.broadcasted_iota(jnp.int32, t.shape, 1)
        d = jnp.where(rid == cid, 1.0, b)
        out_ref[...] = ((dc_ref[...] * d) * dr_ref[...]).astype(BF16)

    return pl.pallas_call(
        body,
        grid=(N // bj,),
        in_specs=[
            pl.BlockSpec((N, bj), lambda j: (0, j)),
            pl.BlockSpec((N, 1), lambda j: (0, 0)),
            pl.BlockSpec((1, bj), lambda j: (0, j)),
        ],
        out_specs=pl.BlockSpec((N, bj), lambda j: (0, j)),
        out_shape=jax.ShapeDtypeStruct((N, N), BF16),
    )(adj, dinv_c, dinv_r)


def _ptmm(p, ys, bias=None, act=None, out_dtype=F32, bm=512, bn=512):
    nd = len(ys)
    nc = ys[0].shape[1]
    bn = min(bn, nc)
    grid = (N // bm, nc // bn)

    def body(p_ref, *rest):
        y_refs = rest[:nd]
        rest = rest[nd:]
        if bias is not None:
            bias_ref = rest[0]
            rest = rest[1:]
        o_refs = rest
        s = p_ref[...]
        for yr, orf in zip(y_refs, o_refs):
            r = lax.dot_general(s, yr[...], _TN, preferred_element_type=F32)
            if bias is not None:
                r = r + bias_ref[...]
            if act is not None:
                r = act(r)
            orf[...] = r.astype(out_dtype)

    in_specs = [pl.BlockSpec((N, bm), lambda m, n: (0, m))]
    in_specs += [pl.BlockSpec((N, bn), lambda m, n: (0, n))] * nd
    args = [p] + list(ys)
    if bias is not None:
        in_specs.append(pl.BlockSpec((1, bn), lambda m, n: (0, n)))
        args.append(bias)
    return pl.pallas_call(
        body,
        grid=grid,
        in_specs=in_specs,
        out_specs=[pl.BlockSpec((bm, bn), lambda m, n: (m, n))] * nd,
        out_shape=[jax.ShapeDtypeStruct((N, nc), out_dtype)] * nd,
        compiler_params=_PARAMS,
    )(*args)


def _mm(avs, bmat, bias=None, act=None, out_dtype=F32, bm=512, bn=512):
    nd = len(avs)
    m_, k_ = avs[0].shape
    nc = bmat.shape[1]
    bn = min(bn, -(-nc // 128) * 128)
    grid = (m_ // bm, -(-nc // bn))
    k_main = (k_ // 128) * 128
    ragged = k_main != k_
    ktail_blk = k_main // 128

    def body(*refs):
        refs = list(refs)
        a_refs = [refs.pop(0) for _ in range(nd)]
        if ragged:
            at_refs = [refs.pop(0) for _ in range(nd)]
        b_ref = refs.pop(0)
        if ragged:
            bt_ref = refs.pop(0)
        if bias is not None:
            bias_ref = refs.pop(0)
        o_refs = refs
        bmain = b_ref[...].astype(BF16)
        if ragged:
            kid = lax.broadcasted_iota(jnp.int32, (128, bn), 0)
            btail = jnp.where(kid < (k_ - k_main), bt_ref[...], 0.0)
            btail = btail.astype(BF16)
        for i in range(nd):
            r = lax.dot_general(a_refs[i][...].astype(BF16), bmain, _NN,
                                preferred_element_type=F32)
            if ragged:
                kida = lax.broadcasted_iota(jnp.int32, (bm, 128), 1)
                atail = jnp.where(kida < (k_ - k_main), at_refs[i][...], 0.0)
                r = r + lax.dot_general(atail.astype(BF16), btail, _NN,
                                        preferred_element_type=F32)
            if bias is not None:
                r = r + bias_ref[...]
            if act is not None:
                r = act(r)
            o_refs[i][...] = r.astype(out_dtype)

    in_specs = [pl.BlockSpec((bm, k_main), lambda m, n: (m, 0))] * nd
    args = list(avs)
    if ragged:
        in_specs += [pl.BlockSpec((bm, 128),
                                  lambda m, n: (m, ktail_blk))] * nd
        args += list(avs)
    in_specs.append(pl.BlockSpec((k_main, bn), lambda m, n: (0, n)))
    args.append(bmat)
    if ragged:
        in_specs.append(pl.BlockSpec((128, bn), lambda m, n: (ktail_blk, n)))
        args.append(bmat)
    if bias is not None:
        in_specs.append(pl.BlockSpec((1, bn), lambda m, n: (0, n)))
        args.append(bias)
    return pl.pallas_call(
        body,
        grid=grid,
        in_specs=in_specs,
        out_specs=[pl.BlockSpec((bm, bn), lambda m, n: (m, n))] * nd,
        out_shape=[jax.ShapeDtypeStruct((m_, nc), out_dtype)] * nd,
        compiler_params=_PARAMS,
    )(*args)


def _vsum_readout_disc(mask, e1, e2, wd, bd):
    bmr = 512

    def body(mask_ref, e1f_ref, e2f_ref, e1r_ref, e2r_ref, wd_ref, bd_ref,
             ret_ref, reta_ref):
        mk = mask_ref[...]
        rs = jnp.sum(mk, axis=1, keepdims=True)
        mb = mk.astype(BF16)
        vs1 = lax.dot_general(mb, e1f_ref[...].astype(BF16), _NN,
                              preferred_element_type=F32)
        vs2 = lax.dot_general(mb, e2f_ref[...].astype(BF16), _NN,
                              preferred_element_type=F32)
        gp = vs1 / rs
        gap = vs2 / rs
        n1 = jnp.maximum(jnp.sqrt(jnp.sum(gp * gp, axis=1, keepdims=True)), 1e-12)
        n2 = jnp.maximum(jnp.sqrt(jnp.sum(gap * gap, axis=1, keepdims=True)), 1e-12)
        g = jax.nn.sigmoid(gp / n1)
        ga = jax.nn.sigmoid(gap / n2)
        w = wd_ref[...]
        u = jnp.dot(e1r_ref[...], w, preferred_element_type=F32)
        ua = jnp.dot(e2r_ref[...], w, preferred_element_type=F32)
        bdv = bd_ref[0, 0]
        ret_ref[...] = jnp.concatenate([
            jnp.sum(u * g, axis=1, keepdims=True) + bdv,
            jnp.sum(ua * g, axis=1, keepdims=True) + bdv], axis=1)
        reta_ref[...] = jnp.concatenate([
            jnp.sum(ua * ga, axis=1, keepdims=True) + bdv,
            jnp.sum(u * ga, axis=1, keepdims=True) + bdv], axis=1)

    return pl.pallas_call(
        body,
        grid=(N // bmr,),
        in_specs=[
            pl.BlockSpec((bmr, N), lambda m: (m, 0)),
            pl.BlockSpec((N, OUT), lambda m: (0, 0)),
            pl.BlockSpec((N, OUT), lambda m: (0, 0)),
            pl.BlockSpec((bmr, OUT), lambda m: (m, 0)),
            pl.BlockSpec((bmr, OUT), lambda m: (m, 0)),
            pl.BlockSpec((OUT, OUT), lambda m: (0, 0)),
            pl.BlockSpec((1, 1), lambda m: (0, 0)),
        ],
        out_specs=[
            pl.BlockSpec((bmr, 2), lambda m: (m, 0)),
            pl.BlockSpec((bmr, 2), lambda m: (m, 0)),
        ],
        out_shape=[
            jax.ShapeDtypeStruct((N, 2), F32),
            jax.ShapeDtypeStruct((N, 2), F32),
        ],
    )(mask, e1, e2, e1, e2, wd, bd)


def _attention(e1, e2, aw, u_row):
    bmr = 256

    def body(e1_ref, e2_ref, aw_ref, u_ref, alpha_ref, comb_ref):
        x1 = e1_ref[...]
        x2 = e2_ref[...]
        w = aw_ref[...]
        u = u_ref[...]
        v1 = jnp.tanh(jnp.dot(x1, w, preferred_element_type=F32))
        v2 = jnp.tanh(jnp.dot(x2, w, preferred_element_type=F32))
        s1 = jnp.sum(v1 * u, axis=1, keepdims=True) + 1e-6
        s2 = jnp.sum(v2 * u, axis=1, keepdims=True) + 1e-6
        mx = jnp.maximum(s1, s2)
        p1 = jnp.exp(s1 - mx)
        p2 = jnp.exp(s2 - mx)
        den = p1 + p2
        a1 = p1 / den
        a2 = p2 / den
        alpha_ref[...] = jnp.concatenate([a1, a2], axis=1)
        comb_ref[...] = a1 * x1 + a2 * x2

    return pl.pallas_call(
        body,
        grid=(N // bmr,),
        in_specs=[
            pl.BlockSpec((bmr, OUT), lambda m: (m, 0)),
            pl.BlockSpec((bmr, OUT), lambda m: (m, 0)),
            pl.BlockSpec((OUT, OUT), lambda m: (0, 0)),
            pl.BlockSpec((1, OUT), lambda m: (0, 0)),
        ],
        out_specs=[
            pl.BlockSpec((bmr, 2), lambda m: (m, 0)),
            pl.BlockSpec((bmr, OUT), lambda m: (m, 0)),
        ],
        out_shape=[
            jax.ShapeDtypeStruct((N, 2), F32),
            jax.ShapeDtypeStruct((N, OUT), F32),
        ],
    )(e1, e2, aw, u_row)


def _encode(adj, feat, feat_sh, neigh, w1, b1, w2, b2, wd, bd):
    dinv = _dinv(adj)
    pmat = _norm(adj, dinv.reshape(N, 1), dinv.reshape(1, N))
    ya, yb = _mm([feat, feat_sh], w1, out_dtype=BF16)
    z1, z2 = _ptmm(pmat, [ya, yb], bias=b1.reshape(1, HID),
                   act=jax.nn.relu, out_dtype=BF16)
    p1, p2 = _mm([z1, z2], w2, out_dtype=BF16)
    e1, e2 = _ptmm(pmat, [p1, p2], bias=b2.reshape(1, OUT))
    ret, ret_a = _vsum_readout_disc(neigh, e1, e2, wd, bd.reshape(1, 1))
    return pmat, e1, ret, ret_a


def _decode(pmat, comb, w1, b1, w2, b2):
    (c1,) = _mm([comb], w1, out_dtype=BF16)
    (h,) = _ptmm(pmat, [c1], bias=b1.reshape(1, -1),
                 act=jax.nn.relu, out_dtype=BF16)
    (g,) = _ptmm(pmat, [h], out_dtype=BF16)
    (rec,) = _mm([g], w2, bias=b2.reshape(1, -1))
    return rec


def kernel(omics1_feat_shuffle, omics2_feat_shuffle, omics1_feat, omics2_feat,
           omics1_adj, omics2_adj, omics1_graph_neigh, omics2_graph_neigh,
           o1_enc_W1, o1_enc_b1, o1_enc_W2, o1_enc_b2, o1_disc_W, o1_disc_b,
           o2_enc_W1, o2_enc_b1, o2_enc_W2, o2_enc_b2, o2_disc_W, o2_disc_b,
           o1_dec_W1, o1_dec_b1, o1_dec_W2, o1_dec_b2,
           o2_dec_W1, o2_dec_b1, o2_dec_W2, o2_dec_b2, att_w, att_u):
    p1m, o1_emb, o1_ret, o1_ret_a = _encode(
        omics1_adj, omics1_feat, omics1_feat_shuffle, omics1_graph_neigh,
        o1_enc_W1, o1_enc_b1, o1_enc_W2, o1_enc_b2, o1_disc_W, o1_disc_b)
    p2m, o2_emb, o2_ret, o2_ret_a = _encode(
        omics2_adj, omics2_feat, omics2_feat_shuffle, omics2_graph_neigh,
        o2_enc_W1, o2_enc_b1, o2_enc_W2, o2_enc_b2, o2_disc_W, o2_disc_b)
    alpha, comb = _attention(o1_emb, o2_emb, att_w, att_u.reshape(1, OUT))
    o1_rec = _decode(p1m, comb, o1_dec_W1, o1_dec_b1, o1_dec_W2, o1_dec_b2)
    o2_rec = _decode(p2m, comb, o2_dec_W1, o2_dec_b1, o2_dec_W2, o2_dec_b2)
    return (o1_emb, o1_rec, o1_ret, o1_ret_a,
            o2_emb, o2_rec, o2_ret, o2_ret_a, comb, alpha)


# speedup vs baseline: 1.0952x; 1.0952x over previous
import jax
import jax.numpy as jnp
from jax import lax
from jax.experimental import pallas as pl
from jax.experimental.pallas import tpu as pltpu

N = 2048
HID = 256
OUT = 128
F32 = jnp.float32
BF16 = jnp.bfloat16

_TN = (((0,), (0,)), ((), ()))
_NN = (((1,), (0,)), ((), ()))
_PARAMS = pltpu.CompilerParams(
    dimension_semantics=("parallel", "parallel"))


def _dinv2(adj1, adj2):
    bj = 512

    def body(a1_ref, a2_ref, o1_ref, o2_ref):
        j = pl.program_id(0)
        for a_ref, o_ref in ((a1_ref, o1_ref), (a2_ref, o2_ref)):
            t = a_ref[...]
            b = jnp.where(t != 0, 1.0, 0.0)
            rid = lax.broadcasted_iota(jnp.int32, t.shape, 0)
            cid = j * bj + lax.broadcasted_iota(jnp.int32, t.shape, 1)
            d = jnp.where(rid == cid, 1.0, b)
            o_ref[...] = 1.0 / jnp.sqrt(jnp.sum(d, axis=0))

    return pl.pallas_call(
        body,
        grid=(N // bj,),
        in_specs=[pl.BlockSpec((N, bj), lambda j: (0, j))] * 2,
        out_specs=[pl.BlockSpec((bj,), lambda j: (j,))] * 2,
        out_shape=[jax.ShapeDtypeStruct((N,), F32)] * 2,
    )(adj1, adj2)


def _norm2(adj1, d1, adj2, d2):
    bj = 512

    def body(a1_ref, d1c_ref, d1r_ref, a2_ref, d2c_ref, d2r_ref,
             o1_ref, o2_ref):
        j = pl.program_id(0)
        for a_ref, dc_ref, dr_ref, o_ref in (
                (a1_ref, d1c_ref, d1r_ref, o1_ref),
                (a2_ref, d2c_ref, d2r_ref, o2_ref)):
            t = a_ref[...]
            b = jnp.where(t != 0, 1.0, 0.0)
            rid = lax.broadcasted_iota(jnp.int32, t.shape, 0)
            cid = j * bj + lax.broadcasted_iota(jnp.int32, t.shape, 1)
            d = jnp.where(rid == cid, 1.0, b)
            o_ref[...] = ((dc_ref[...] * d) * dr_ref[...]).astype(BF16)

    specs_one = [
        pl.BlockSpec((N, bj), lambda j: (0, j)),
        pl.BlockSpec((N, 1), lambda j: (0, 0)),
        pl.BlockSpec((1, bj), lambda j: (0, j)),
    ]
    return pl.pallas_call(
        body,
        grid=(N // bj,),
        in_specs=specs_one + specs_one,
        out_specs=[pl.BlockSpec((N, bj), lambda j: (0, j))] * 2,
        out_shape=[jax.ShapeDtypeStruct((N, N), BF16)] * 2,
    )(adj1, d1.reshape(N, 1), d1.reshape(1, N),
      adj2, d2.reshape(N, 1), d2.reshape(1, N))


def _ptmm2(ps, yss, biases=None, act=None, out_dtype=F32, bm=512):
    ng = len(ps)
    counts = [len(ys) for ys in yss]
    ncs = [ys[0].shape[1] for ys in yss]

    def body(*refs):
        refs = list(refs)
        p_refs = [refs.pop(0) for _ in range(ng)]
        y_refs = [[refs.pop(0) for _ in range(counts[g])] for g in range(ng)]
        b_refs = ([refs.pop(0) for _ in range(ng)]
                  if biases is not None else [None] * ng)
        for g in range(ng):
            s = p_refs[g][...]
            for i in range(counts[g]):
                r = lax.dot_general(s, y_refs[g][i][...], _TN,
                                    preferred_element_type=F32)
                if biases is not None:
                    r = r + b_refs[g][...]
                if act is not None:
                    r = act(r)
                refs.pop(0)[...] = r.astype(out_dtype)

    in_specs = [pl.BlockSpec((N, bm), lambda m: (0, m))] * ng
    args = list(ps)
    for g in range(ng):
        in_specs += [pl.BlockSpec((N, ncs[g]), lambda m: (0, 0))] * counts[g]
        args += list(yss[g])
    if biases is not None:
        for g in range(ng):
            in_specs.append(pl.BlockSpec((1, ncs[g]), lambda m: (0, 0)))
            args.append(biases[g].reshape(1, ncs[g]))
    out_specs = []
    out_shape = []
    for g in range(ng):
        out_specs += [pl.BlockSpec((bm, ncs[g]), lambda m: (m, 0))] * counts[g]
        out_shape += [jax.ShapeDtypeStruct((N, ncs[g]), out_dtype)] * counts[g]
    flat = pl.pallas_call(
        body,
        grid=(N // bm,),
        in_specs=in_specs,
        out_specs=out_specs,
        out_shape=out_shape,
    )(*args)
    outs = []
    k = 0
    for g in range(ng):
        outs.append(list(flat[k:k + counts[g]]))
        k += counts[g]
    return outs


def _mm(avs, bmat, bias=None, act=None, out_dtype=F32, bm=512, bn=512):
    nd = len(avs)
    m_, k_ = avs[0].shape
    nc = bmat.shape[1]
    bn = min(bn, -(-nc // 128) * 128)
    grid = (m_ // bm, -(-nc // bn))
    k_main = (k_ // 128) * 128
    ragged = k_main != k_
    ktail_blk = k_main // 128

    def body(*refs):
        refs = list(refs)
        a_refs = [refs.pop(0) for _ in range(nd)]
        if ragged:
            at_refs = [refs.pop(0) for _ in range(nd)]
        b_ref = refs.pop(0)
        if ragged:
            bt_ref = refs.pop(0)
        if bias is not None:
            bias_ref = refs.pop(0)
        o_refs = refs
        bmain = b_ref[...].astype(BF16)
        if ragged:
            kid = lax.broadcasted_iota(jnp.int32, (128, bn), 0)
            btail = jnp.where(kid < (k_ - k_main), bt_ref[...], 0.0)
            btail = btail.astype(BF16)
        for i in range(nd):
            r = lax.dot_general(a_refs[i][...].astype(BF16), bmain, _NN,
                                preferred_element_type=F32)
            if ragged:
                kida = lax.broadcasted_iota(jnp.int32, (bm, 128), 1)
                atail = jnp.where(kida < (k_ - k_main), at_refs[i][...], 0.0)
                r = r + lax.dot_general(atail.astype(BF16), btail, _NN,
                                        preferred_element_type=F32)
            if bias is not None:
                r = r + bias_ref[...]
            if act is not None:
                r = act(r)
            o_refs[i][...] = r.astype(out_dtype)

    in_specs = [pl.BlockSpec((bm, k_main), lambda m, n: (m, 0))] * nd
    args = list(avs)
    if ragged:
        in_specs += [pl.BlockSpec((bm, 128),
                                  lambda m, n: (m, ktail_blk))] * nd
        args += list(avs)
    in_specs.append(pl.BlockSpec((k_main, bn), lambda m, n: (0, n)))
    args.append(bmat)
    if ragged:
        in_specs.append(pl.BlockSpec((128, bn), lambda m, n: (ktail_blk, n)))
        args.append(bmat)
    if bias is not None:
        in_specs.append(pl.BlockSpec((1, bn), lambda m, n: (0, n)))
        args.append(bias)
    return pl.pallas_call(
        body,
        grid=grid,
        in_specs=in_specs,
        out_specs=[pl.BlockSpec((bm, bn), lambda m, n: (m, n))] * nd,
        out_shape=[jax.ShapeDtypeStruct((m_, nc), out_dtype)] * nd,
        compiler_params=_PARAMS,
    )(*args)


def _mm_two_rhs(a, b1, b2, out_dtype=F32, bm=512):
    k_ = a.shape[1]
    nc1 = b1.shape[1]
    nc2 = b2.shape[1]

    def body(a_ref, b1_ref, b2_ref, o1_ref, o2_ref):
        av = a_ref[...].astype(BF16)
        o1_ref[...] = lax.dot_general(
            av, b1_ref[...].astype(BF16), _NN,
            preferred_element_type=F32).astype(out_dtype)
        o2_ref[...] = lax.dot_general(
            av, b2_ref[...].astype(BF16), _NN,
            preferred_element_type=F32).astype(out_dtype)

    return pl.pallas_call(
        body,
        grid=(N // bm,),
        in_specs=[
            pl.BlockSpec((bm, k_), lambda m: (m, 0)),
            pl.BlockSpec((k_, nc1), lambda m: (0, 0)),
            pl.BlockSpec((k_, nc2), lambda m: (0, 0)),
        ],
        out_specs=[
            pl.BlockSpec((bm, nc1), lambda m: (m, 0)),
            pl.BlockSpec((bm, nc2), lambda m: (m, 0)),
        ],
        out_shape=[
            jax.ShapeDtypeStruct((N, nc1), out_dtype),
            jax.ShapeDtypeStruct((N, nc2), out_dtype),
        ],
    )(a, b1, b2)


def _readout_disc2(groups):
    bmr = 512

    def one(mk, ef, eaf, er, ear, w, bdv, ret_ref, reta_ref):
        rs = jnp.sum(mk, axis=1, keepdims=True)
        mb = mk.astype(BF16)
        vs1 = lax.dot_general(mb, ef.astype(BF16), _NN,
                              preferred_element_type=F32)
        vs2 = lax.dot_general(mb, eaf.astype(BF16), _NN,
                              preferred_element_type=F32)
        gp = vs1 / rs
        gap = vs2 / rs
        n1 = jnp.maximum(jnp.sqrt(jnp.sum(gp * gp, axis=1, keepdims=True)),
                         1e-12)
        n2 = jnp.maximum(jnp.sqrt(jnp.sum(gap * gap, axis=1, keepdims=True)),
                         1e-12)
        g = jax.nn.sigmoid(gp / n1)
        ga = jax.nn.sigmoid(gap / n2)
        u = jnp.dot(er, w, preferred_element_type=F32)
        ua = jnp.dot(ear, w, preferred_element_type=F32)
        ret_ref[...] = jnp.concatenate([
            jnp.sum(u * g, axis=1, keepdims=True) + bdv,
            jnp.sum(ua * g, axis=1, keepdims=True) + bdv], axis=1)
        reta_ref[...] = jnp.concatenate([
            jnp.sum(ua * ga, axis=1, keepdims=True) + bdv,
            jnp.sum(u * ga, axis=1, keepdims=True) + bdv], axis=1)

    def body(m1, e1f, e1af, e1r, e1ar, w1, b1,
             m2, e2f, e2af, e2r, e2ar, w2, b2,
             ret1, reta1, ret2, reta2):
        one(m1[...], e1f[...], e1af[...], e1r[...], e1ar[...], w1[...],
            b1[0, 0], ret1, reta1)
        one(m2[...], e2f[...], e2af[...], e2r[...], e2ar[...], w2[...],
            b2[0, 0], ret2, reta2)

    specs_one = [
        pl.BlockSpec((bmr, N), lambda m: (m, 0)),
        pl.BlockSpec((N, OUT), lambda m: (0, 0)),
        pl.BlockSpec((N, OUT), lambda m: (0, 0)),
        pl.BlockSpec((bmr, OUT), lambda m: (m, 0)),
        pl.BlockSpec((bmr, OUT), lambda m: (m, 0)),
        pl.BlockSpec((OUT, OUT), lambda m: (0, 0)),
        pl.BlockSpec((1, 1), lambda m: (0, 0)),
    ]
    args = []
    for mask, ec, es, wd, bd in groups:
        args += [mask, ec, es, ec, es, wd, bd.reshape(1, 1)]
    return pl.pallas_call(
        body,
        grid=(N // bmr,),
        in_specs=specs_one + specs_one,
        out_specs=[pl.BlockSpec((bmr, 2), lambda m: (m, 0))] * 4,
        out_shape=[jax.ShapeDtypeStruct((N, 2), F32)] * 4,
    )(*args)


def _attention(e1, e2, aw, u_row):
    bmr = 256

    def body(e1_ref, e2_ref, aw_ref, u_ref, alpha_ref, comb_ref):
        x1 = e1_ref[...]
        x2 = e2_ref[...]
        w = aw_ref[...]
        u = u_ref[...]
        v1 = jnp.tanh(jnp.dot(x1, w, preferred_element_type=F32))
        v2 = jnp.tanh(jnp.dot(x2, w, preferred_element_type=F32))
        s1 = jnp.sum(v1 * u, axis=1, keepdims=True) + 1e-6
        s2 = jnp.sum(v2 * u, axis=1, keepdims=True) + 1e-6
        mx = jnp.maximum(s1, s2)
        p1 = jnp.exp(s1 - mx)
        p2 = jnp.exp(s2 - mx)
        den = p1 + p2
        a1 = p1 / den
        a2 = p2 / den
        alpha_ref[...] = jnp.concatenate([a1, a2], axis=1)
        comb_ref[...] = a1 * x1 + a2 * x2

    return pl.pallas_call(
        body,
        grid=(N // bmr,),
        in_specs=[
            pl.BlockSpec((bmr, OUT), lambda m: (m, 0)),
            pl.BlockSpec((bmr, OUT), lambda m: (m, 0)),
            pl.BlockSpec((OUT, OUT), lambda m: (0, 0)),
            pl.BlockSpec((1, OUT), lambda m: (0, 0)),
        ],
        out_specs=[
            pl.BlockSpec((bmr, 2), lambda m: (m, 0)),
            pl.BlockSpec((bmr, OUT), lambda m: (m, 0)),
        ],
        out_shape=[
            jax.ShapeDtypeStruct((N, 2), F32),
            jax.ShapeDtypeStruct((N, OUT), F32),
        ],
    )(e1, e2, aw, u_row)


def kernel(omics1_feat_shuffle, omics2_feat_shuffle, omics1_feat, omics2_feat,
           omics1_adj, omics2_adj, omics1_graph_neigh, omics2_graph_neigh,
           o1_enc_W1, o1_enc_b1, o1_enc_W2, o1_enc_b2, o1_disc_W, o1_disc_b,
           o2_enc_W1, o2_enc_b1, o2_enc_W2, o2_enc_b2, o2_disc_W, o2_disc_b,
           o1_dec_W1, o1_dec_b1, o1_dec_W2, o1_dec_b2,
           o2_dec_W1, o2_dec_b1, o2_dec_W2, o2_dec_b2, att_w, att_u):
    d1, d2 = _dinv2(omics1_adj, omics2_adj)
    p1m, p2m = _norm2(omics1_adj, d1, omics2_adj, d2)

    y1a, y1b = _mm([omics1_feat, omics1_feat_shuffle], o1_enc_W1,
                   out_dtype=BF16)
    y2a, y2b = _mm([omics2_feat, omics2_feat_shuffle], o2_enc_W1,
                   out_dtype=BF16)

    (z1a, z1b), (z2a, z2b) = _ptmm2(
        [p1m, p2m], [[y1a, y1b], [y2a, y2b]],
        biases=[o1_enc_b1, o2_enc_b1], act=jax.nn.relu, out_dtype=BF16)

    q1a, q1b = _mm([z1a, z1b], o1_enc_W2, out_dtype=BF16)
    q2a, q2b = _mm([z2a, z2b], o2_enc_W2, out_dtype=BF16)

    (e1a, e1b), (e2a, e2b) = _ptmm2(
        [p1m, p2m], [[q1a, q1b], [q2a, q2b]],
        biases=[o1_enc_b2, o2_enc_b2])

    o1_ret, o1_ret_a, o2_ret, o2_ret_a = _readout_disc2([
        (omics1_graph_neigh, e1a, e1b, o1_disc_W, o1_disc_b),
        (omics2_graph_neigh, e2a, e2b, o2_disc_W, o2_disc_b)])

    alpha, comb = _attention(e1a, e2a, att_w, att_u.reshape(1, OUT))

    c1, c2 = _mm_two_rhs(comb, o1_dec_W1, o2_dec_W1, out_dtype=BF16)
    (h1,), (h2,) = _ptmm2(
        [p1m, p2m], [[c1], [c2]],
        biases=[o1_dec_b1, o2_dec_b1], act=jax.nn.relu, out_dtype=BF16)
    (g1,), (g2,) = _ptmm2([p1m, p2m], [[h1], [h2]], out_dtype=BF16)
    (o1_rec,) = _mm([g1], o1_dec_W2, bias=o1_dec_b2.reshape(1, -1))
    (o2_rec,) = _mm([g2], o2_dec_W2, bias=o2_dec_b2.reshape(1, -1))

    return (e1a, o1_rec, o1_ret, o1_ret_a,
            e2a, o2_rec, o2_ret, o2_ret_a, comb, alpha)


# speedup vs baseline: 1.1749x; 1.0728x over previous
import jax
import jax.numpy as jnp
from jax import lax
from jax.experimental import pallas as pl
from jax.experimental.pallas import tpu as pltpu

N = 2048
HID = 256
OUT = 128
F32 = jnp.float32
BF16 = jnp.bfloat16

_TN = (((0,), (0,)), ((), ()))
_NN = (((1,), (0,)), ((), ()))
_PARAMS = pltpu.CompilerParams(
    dimension_semantics=("parallel", "parallel"))


def _dinv2(adj1, adj2):
    bj = 512

    def body(a1_ref, a2_ref, o1_ref, o2_ref):
        j = pl.program_id(0)
        for a_ref, o_ref in ((a1_ref, o1_ref), (a2_ref, o2_ref)):
            t = a_ref[...]
            b = jnp.where(t != 0, 1.0, 0.0)
            rid = lax.broadcasted_iota(jnp.int32, t.shape, 0)
            cid = j * bj + lax.broadcasted_iota(jnp.int32, t.shape, 1)
            d = jnp.where(rid == cid, 1.0, b)
            o_ref[...] = 1.0 / jnp.sqrt(jnp.sum(d, axis=0))

    return pl.pallas_call(
        body,
        grid=(N // bj,),
        in_specs=[pl.BlockSpec((N, bj), lambda j: (0, j))] * 2,
        out_specs=[pl.BlockSpec((bj,), lambda j: (j,))] * 2,
        out_shape=[jax.ShapeDtypeStruct((N,), F32)] * 2,
    )(adj1, adj2)


def _norm2(adj1, d1, adj2, d2):
    bj = 512

    def body(a1_ref, d1c_ref, d1r_ref, a2_ref, d2c_ref, d2r_ref,
             o1_ref, o2_ref):
        j = pl.program_id(0)
        for a_ref, dc_ref, dr_ref, o_ref in (
                (a1_ref, d1c_ref, d1r_ref, o1_ref),
                (a2_ref, d2c_ref, d2r_ref, o2_ref)):
            t = a_ref[...]
            b = jnp.where(t != 0, 1.0, 0.0)
            rid = lax.broadcasted_iota(jnp.int32, t.shape, 0)
            cid = j * bj + lax.broadcasted_iota(jnp.int32, t.shape, 1)
            d = jnp.where(rid == cid, 1.0, b)
            o_ref[...] = ((dc_ref[...] * d) * dr_ref[...]).astype(BF16)

    specs_one = [
        pl.BlockSpec((N, bj), lambda j: (0, j)),
        pl.BlockSpec((N, 1), lambda j: (0, 0)),
        pl.BlockSpec((1, bj), lambda j: (0, j)),
    ]
    return pl.pallas_call(
        body,
        grid=(N // bj,),
        in_specs=specs_one + specs_one,
        out_specs=[pl.BlockSpec((N, bj), lambda j: (0, j))] * 2,
        out_shape=[jax.ShapeDtypeStruct((N, N), BF16)] * 2,
    )(adj1, d1.reshape(N, 1), d1.reshape(1, N),
      adj2, d2.reshape(N, 1), d2.reshape(1, N))


def _ptmm2(ps, yss, biases=None, act=None, out_dtype=F32, bm=512,
           pre_ws=None):
    ng = len(ps)
    counts = [len(ys) for ys in yss]
    kcs = [ys[0].shape[1] for ys in yss]
    if pre_ws is not None:
        ncs = [w.shape[1] for w in pre_ws]
    else:
        ncs = kcs

    def body(*refs):
        refs = list(refs)
        p_refs = [refs.pop(0) for _ in range(ng)]
        y_refs = [[refs.pop(0) for _ in range(counts[g])] for g in range(ng)]
        w_refs = ([refs.pop(0) for _ in range(ng)]
                  if pre_ws is not None else [None] * ng)
        b_refs = ([refs.pop(0) for _ in range(ng)]
                  if biases is not None else [None] * ng)
        for g in range(ng):
            s = p_refs[g][...]
            if pre_ws is not None:
                w = w_refs[g][...].astype(BF16)
            for i in range(counts[g]):
                y = y_refs[g][i][...]
                if pre_ws is not None:
                    y = lax.dot_general(y, w, _NN,
                                        preferred_element_type=F32)
                    y = y.astype(BF16)
                r = lax.dot_general(s, y, _TN,
                                    preferred_element_type=F32)
                if biases is not None:
                    r = r + b_refs[g][...]
                if act is not None:
                    r = act(r)
                refs.pop(0)[...] = r.astype(out_dtype)

    in_specs = [pl.BlockSpec((N, bm), lambda m: (0, m))] * ng
    args = list(ps)
    for g in range(ng):
        in_specs += [pl.BlockSpec((N, kcs[g]), lambda m: (0, 0))] * counts[g]
        args += list(yss[g])
    if pre_ws is not None:
        for g in range(ng):
            in_specs.append(
                pl.BlockSpec((kcs[g], ncs[g]), lambda m: (0, 0)))
            args.append(pre_ws[g])
    if biases is not None:
        for g in range(ng):
            in_specs.append(pl.BlockSpec((1, ncs[g]), lambda m: (0, 0)))
            args.append(biases[g].reshape(1, ncs[g]))
    out_specs = []
    out_shape = []
    for g in range(ng):
        out_specs += [pl.BlockSpec((bm, ncs[g]), lambda m: (m, 0))] * counts[g]
        out_shape += [jax.ShapeDtypeStruct((N, ncs[g]), out_dtype)] * counts[g]
    flat = pl.pallas_call(
        body,
        grid=(N // bm,),
        in_specs=in_specs,
        out_specs=out_specs,
        out_shape=out_shape,
    )(*args)
    outs = []
    k = 0
    for g in range(ng):
        outs.append(list(flat[k:k + counts[g]]))
        k += counts[g]
    return outs


def _mm(avs, bmat, bias=None, act=None, out_dtype=F32, bm=512, bn=512):
    nd = len(avs)
    m_, k_ = avs[0].shape
    nc = bmat.shape[1]
    bn = min(bn, -(-nc // 128) * 128)
    grid = (m_ // bm, -(-nc // bn))
    k_main = (k_ // 128) * 128
    ragged = k_main != k_
    ktail_blk = k_main // 128

    def body(*refs):
        refs = list(refs)
        a_refs = [refs.pop(0) for _ in range(nd)]
        if ragged:
            at_refs = [refs.pop(0) for _ in range(nd)]
        b_ref = refs.pop(0)
        if ragged:
            bt_ref = refs.pop(0)
        if bias is not None:
            bias_ref = refs.pop(0)
        o_refs = refs
        bmain = b_ref[...].astype(BF16)
        if ragged:
            kid = lax.broadcasted_iota(jnp.int32, (128, bn), 0)
            btail = jnp.where(kid < (k_ - k_main), bt_ref[...], 0.0)
            btail = btail.astype(BF16)
        for i in range(nd):
            r = lax.dot_general(a_refs[i][...].astype(BF16), bmain, _NN,
                                preferred_element_type=F32)
            if ragged:
                kida = lax.broadcasted_iota(jnp.int32, (bm, 128), 1)
                atail = jnp.where(kida < (k_ - k_main), at_refs[i][...], 0.0)
                r = r + lax.dot_general(atail.astype(BF16), btail, _NN,
                                        preferred_element_type=F32)
            if bias is not None:
                r = r + bias_ref[...]
            if act is not None:
                r = act(r)
            o_refs[i][...] = r.astype(out_dtype)

    in_specs = [pl.BlockSpec((bm, k_main), lambda m, n: (m, 0))] * nd
    args = list(avs)
    if ragged:
        in_specs += [pl.BlockSpec((bm, 128),
                                  lambda m, n: (m, ktail_blk))] * nd
        args += list(avs)
    in_specs.append(pl.BlockSpec((k_main, bn), lambda m, n: (0, n)))
    args.append(bmat)
    if ragged:
        in_specs.append(pl.BlockSpec((128, bn), lambda m, n: (ktail_blk, n)))
        args.append(bmat)
    if bias is not None:
        in_specs.append(pl.BlockSpec((1, bn), lambda m, n: (0, n)))
        args.append(bias)
    return pl.pallas_call(
        body,
        grid=grid,
        in_specs=in_specs,
        out_specs=[pl.BlockSpec((bm, bn), lambda m, n: (m, n))] * nd,
        out_shape=[jax.ShapeDtypeStruct((m_, nc), out_dtype)] * nd,
        compiler_params=_PARAMS,
    )(*args)


def _readout_disc_attn(groups, aw, u_row, dec_w1s):
    bmr = 512
    nh = dec_w1s[0].shape[1]

    def one(mk, ef, eaf, er, ear, w, bdv, ret_ref, reta_ref):
        rs = jnp.sum(mk, axis=1, keepdims=True)
        mb = mk.astype(BF16)
        vs1 = lax.dot_general(mb, ef.astype(BF16), _NN,
                              preferred_element_type=F32)
        vs2 = lax.dot_general(mb, eaf.astype(BF16), _NN,
                              preferred_element_type=F32)
        gp = vs1 / rs
        gap = vs2 / rs
        n1 = jnp.maximum(jnp.sqrt(jnp.sum(gp * gp, axis=1, keepdims=True)),
                         1e-12)
        n2 = jnp.maximum(jnp.sqrt(jnp.sum(gap * gap, axis=1, keepdims=True)),
                         1e-12)
        g = jax.nn.sigmoid(gp / n1)
        ga = jax.nn.sigmoid(gap / n2)
        u = jnp.dot(er, w, preferred_element_type=F32)
        ua = jnp.dot(ear, w, preferred_element_type=F32)
        ret_ref[...] = jnp.concatenate([
            jnp.sum(u * g, axis=1, keepdims=True) + bdv,
            jnp.sum(ua * g, axis=1, keepdims=True) + bdv], axis=1)
        reta_ref[...] = jnp.concatenate([
            jnp.sum(ua * ga, axis=1, keepdims=True) + bdv,
            jnp.sum(u * ga, axis=1, keepdims=True) + bdv], axis=1)

    def body(m1, e1f, e1af, e1r, e1ar, w1, b1,
             m2, e2f, e2af, e2r, e2ar, w2, b2,
             aw_ref, u_ref, dw1_ref, dw2_ref,
             ret1, reta1, ret2, reta2, alpha_ref, comb_ref, c1_ref, c2_ref):
        one(m1[...], e1f[...], e1af[...], e1r[...], e1ar[...], w1[...],
            b1[0, 0], ret1, reta1)
        one(m2[...], e2f[...], e2af[...], e2r[...], e2ar[...], w2[...],
            b2[0, 0], ret2, reta2)
        x1 = e1r[...]
        x2 = e2r[...]
        w = aw_ref[...]
        u = u_ref[...]
        v1 = jnp.tanh(jnp.dot(x1, w, preferred_element_type=F32))
        v2 = jnp.tanh(jnp.dot(x2, w, preferred_element_type=F32))
        s1 = jnp.sum(v1 * u, axis=1, keepdims=True) + 1e-6
        s2 = jnp.sum(v2 * u, axis=1, keepdims=True) + 1e-6
        mx = jnp.maximum(s1, s2)
        p1 = jnp.exp(s1 - mx)
        p2 = jnp.exp(s2 - mx)
        den = p1 + p2
        a1 = p1 / den
        a2 = p2 / den
        alpha_ref[...] = jnp.concatenate([a1, a2], axis=1)
        comb = a1 * x1 + a2 * x2
        comb_ref[...] = comb
        cb = comb.astype(BF16)
        c1_ref[...] = lax.dot_general(
            cb, dw1_ref[...].astype(BF16), _NN,
            preferred_element_type=F32).astype(BF16)
        c2_ref[...] = lax.dot_general(
            cb, dw2_ref[...].astype(BF16), _NN,
            preferred_element_type=F32).astype(BF16)

    specs_one = [
        pl.BlockSpec((bmr, N), lambda m: (m, 0)),
        pl.BlockSpec((N, OUT), lambda m: (0, 0)),
        pl.BlockSpec((N, OUT), lambda m: (0, 0)),
        pl.BlockSpec((bmr, OUT), lambda m: (m, 0)),
        pl.BlockSpec((bmr, OUT), lambda m: (m, 0)),
        pl.BlockSpec((OUT, OUT), lambda m: (0, 0)),
        pl.BlockSpec((1, 1), lambda m: (0, 0)),
    ]
    args = []
    for mask, ec, es, wd, bd in groups:
        args += [mask, ec, es, ec, es, wd, bd.reshape(1, 1)]
    args += [aw, u_row, dec_w1s[0], dec_w1s[1]]
    return pl.pallas_call(
        body,
        grid=(N // bmr,),
        in_specs=specs_one + specs_one + [
            pl.BlockSpec((OUT, OUT), lambda m: (0, 0)),
            pl.BlockSpec((1, OUT), lambda m: (0, 0)),
            pl.BlockSpec((OUT, nh), lambda m: (0, 0)),
            pl.BlockSpec((OUT, nh), lambda m: (0, 0)),
        ],
        out_specs=[pl.BlockSpec((bmr, 2), lambda m: (m, 0))] * 4 + [
            pl.BlockSpec((bmr, 2), lambda m: (m, 0)),
            pl.BlockSpec((bmr, OUT), lambda m: (m, 0)),
            pl.BlockSpec((bmr, nh), lambda m: (m, 0)),
            pl.BlockSpec((bmr, nh), lambda m: (m, 0)),
        ],
        out_shape=[jax.ShapeDtypeStruct((N, 2), F32)] * 4 + [
            jax.ShapeDtypeStruct((N, 2), F32),
            jax.ShapeDtypeStruct((N, OUT), F32),
            jax.ShapeDtypeStruct((N, nh), BF16),
            jax.ShapeDtypeStruct((N, nh), BF16),
        ],
    )(*args)


def kernel(omics1_feat_shuffle, omics2_feat_shuffle, omics1_feat, omics2_feat,
           omics1_adj, omics2_adj, omics1_graph_neigh, omics2_graph_neigh,
           o1_enc_W1, o1_enc_b1, o1_enc_W2, o1_enc_b2, o1_disc_W, o1_disc_b,
           o2_enc_W1, o2_enc_b1, o2_enc_W2, o2_enc_b2, o2_disc_W, o2_disc_b,
           o1_dec_W1, o1_dec_b1, o1_dec_W2, o1_dec_b2,
           o2_dec_W1, o2_dec_b1, o2_dec_W2, o2_dec_b2, att_w, att_u):
    d1, d2 = _dinv2(omics1_adj, omics2_adj)
    p1m, p2m = _norm2(omics1_adj, d1, omics2_adj, d2)

    y1a, y1b = _mm([omics1_feat, omics1_feat_shuffle], o1_enc_W1,
                   out_dtype=BF16)
    y2a, y2b = _mm([omics2_feat, omics2_feat_shuffle], o2_enc_W1,
                   out_dtype=BF16)

    (z1a, z1b), (z2a, z2b) = _ptmm2(
        [p1m, p2m], [[y1a, y1b], [y2a, y2b]],
        biases=[o1_enc_b1, o2_enc_b1], act=jax.nn.relu, out_dtype=BF16)

    (e1a, e1b), (e2a, e2b) = _ptmm2(
        [p1m, p2m], [[z1a, z1b], [z2a, z2b]],
        biases=[o1_enc_b2, o2_enc_b2], pre_ws=[o1_enc_W2, o2_enc_W2])

    (o1_ret, o1_ret_a, o2_ret, o2_ret_a,
     alpha, comb, c1, c2) = _readout_disc_attn(
        [(omics1_graph_neigh, e1a, e1b, o1_disc_W, o1_disc_b),
         (omics2_graph_neigh, e2a, e2b, o2_disc_W, o2_disc_b)],
        att_w, att_u.reshape(1, OUT), [o1_dec_W1, o2_dec_W1])

    (h1,), (h2,) = _ptmm2(
        [p1m, p2m], [[c1], [c2]],
        biases=[o1_dec_b1, o2_dec_b1], act=jax.nn.relu, out_dtype=BF16)
    (g1,), (g2,) = _ptmm2([p1m, p2m], [[h1], [h2]], out_dtype=BF16)
    (o1_rec,) = _mm([g1], o1_dec_W2, bias=o1_dec_b2.reshape(1, -1))
    (o2_rec,) = _mm([g2], o2_dec_W2, bias=o2_dec_b2.reshape(1, -1))

    return (e1a, o1_rec, o1_ret, o1_ret_a,
            e2a, o2_rec, o2_ret, o2_ret_a, comb, alpha)


# speedup vs baseline: 1.2446x; 1.0593x over previous
import jax
import jax.numpy as jnp
from jax import lax
from jax.experimental import pallas as pl
from jax.experimental.pallas import tpu as pltpu

N = 2048
HID = 256
OUT = 128
F32 = jnp.float32
BF16 = jnp.bfloat16

_TN = (((0,), (0,)), ((), ()))
_NN = (((1,), (0,)), ((), ()))
_PARAMS = pltpu.CompilerParams(
    dimension_semantics=("parallel", "parallel"))


def _dinv2(adj1, adj2):
    bj = 512

    def body(a1_ref, a2_ref, o1_ref, o2_ref):
        j = pl.program_id(0)
        for a_ref, o_ref in ((a1_ref, o1_ref), (a2_ref, o2_ref)):
            t = a_ref[...]
            b = jnp.where(t != 0, 1.0, 0.0)
            rid = lax.broadcasted_iota(jnp.int32, t.shape, 0)
            cid = j * bj + lax.broadcasted_iota(jnp.int32, t.shape, 1)
            d = jnp.where(rid == cid, 1.0, b)
            o_ref[...] = 1.0 / jnp.sqrt(jnp.sum(d, axis=0))

    return pl.pallas_call(
        body,
        grid=(N // bj,),
        in_specs=[pl.BlockSpec((N, bj), lambda j: (0, j))] * 2,
        out_specs=[pl.BlockSpec((bj,), lambda j: (j,))] * 2,
        out_shape=[jax.ShapeDtypeStruct((N,), F32)] * 2,
    )(adj1, adj2)


def _norm2(adj1, d1, adj2, d2):
    bj = 512

    def body(a1_ref, d1c_ref, d1r_ref, a2_ref, d2c_ref, d2r_ref,
             o1_ref, o2_ref):
        j = pl.program_id(0)
        for a_ref, dc_ref, dr_ref, o_ref in (
                (a1_ref, d1c_ref, d1r_ref, o1_ref),
                (a2_ref, d2c_ref, d2r_ref, o2_ref)):
            t = a_ref[...]
            b = jnp.where(t != 0, 1.0, 0.0)
            rid = lax.broadcasted_iota(jnp.int32, t.shape, 0)
            cid = j * bj + lax.broadcasted_iota(jnp.int32, t.shape, 1)
            d = jnp.where(rid == cid, 1.0, b)
            o_ref[...] = ((dc_ref[...] * d) * dr_ref[...]).astype(BF16)

    specs_one = [
        pl.BlockSpec((N, bj), lambda j: (0, j)),
        pl.BlockSpec((N, 1), lambda j: (0, 0)),
        pl.BlockSpec((1, bj), lambda j: (0, j)),
    ]
    return pl.pallas_call(
        body,
        grid=(N // bj,),
        in_specs=specs_one + specs_one,
        out_specs=[pl.BlockSpec((N, bj), lambda j: (0, j))] * 2,
        out_shape=[jax.ShapeDtypeStruct((N, N), BF16)] * 2,
    )(adj1, d1.reshape(N, 1), d1.reshape(1, N),
      adj2, d2.reshape(N, 1), d2.reshape(1, N))


def _core(p1, p2, y1a, y1b, y2a, y2b,
          w2a, b2a, wd1, bd1, w2b, b2b, wd2, bd2,
          mask1, mask2, aw, au, dwa, dba, dwb, dbb,
          b1a, b1b):
    bm = 512
    nb = N // bm

    def gate(phx):
        return lambda ph, m: (jnp.where(ph < phx, 0,
                                        jnp.where(ph == phx, m, nb - 1)), 0)

    def const(ph, m):
        return (0, 0)

    p_idx = lambda ph, m: (0, jnp.where(ph == 2, nb - 1, m))

    def readout_one(mk, ef, eaf, er, ear, w, bdv, ret_ref, reta_ref):
        rs = jnp.sum(mk, axis=1, keepdims=True)
        mb = mk.astype(BF16)
        vs1 = lax.dot_general(mb, ef.astype(BF16), _NN,
                              preferred_element_type=F32)
        vs2 = lax.dot_general(mb, eaf.astype(BF16), _NN,
                              preferred_element_type=F32)
        gp = vs1 / rs
        gap = vs2 / rs
        n1 = jnp.maximum(jnp.sqrt(jnp.sum(gp * gp, axis=1, keepdims=True)),
                         1e-12)
        n2 = jnp.maximum(jnp.sqrt(jnp.sum(gap * gap, axis=1, keepdims=True)),
                         1e-12)
        g = jax.nn.sigmoid(gp / n1)
        ga = jax.nn.sigmoid(gap / n2)
        u = jnp.dot(er, w, preferred_element_type=F32)
        ua = jnp.dot(ear, w, preferred_element_type=F32)
        ret_ref[...] = jnp.concatenate([
            jnp.sum(u * g, axis=1, keepdims=True) + bdv,
            jnp.sum(ua * g, axis=1, keepdims=True) + bdv], axis=1)
        reta_ref[...] = jnp.concatenate([
            jnp.sum(ua * ga, axis=1, keepdims=True) + bdv,
            jnp.sum(u * ga, axis=1, keepdims=True) + bdv], axis=1)

    def body(p1_ref, p2_ref, y1a_ref, y1b_ref, y2a_ref, y2b_ref,
             w2a_ref, b2a_ref, wd1_ref, bd1_ref,
             w2b_ref, b2b_ref, wd2_ref, bd2_ref,
             m1_ref, m2_ref, aw_ref, au_ref,
             dwa_ref, dba_ref, dwb_ref, dbb_ref, b1a_ref, b1b_ref,
             e1_o, e2_o, ret1_o, reta1_o, ret2_o, reta2_o,
             alpha_o, comb_o, g1_o, g2_o,
             z1a_s, z1b_s, z2a_s, z2b_s,
             e1a_s, e1b_s, e2a_s, e2b_s, c1_s, c2_s, h1_s, h2_s):
        ph = pl.program_id(0)
        m = pl.program_id(1)
        rows = pl.ds(m * bm, bm)

        @pl.when(ph == 0)
        def _l1():
            for p_ref, ys, b_ref, zs in (
                    (p1_ref, (y1a_ref, y1b_ref), b1a_ref, (z1a_s, z1b_s)),
                    (p2_ref, (y2a_ref, y2b_ref), b1b_ref, (z2a_s, z2b_s))):
                s = p_ref[...]
                for y_ref, z_ref in zip(ys, zs):
                    r = lax.dot_general(s, y_ref[...], _TN,
                                        preferred_element_type=F32)
                    z_ref[rows, :] = jax.nn.relu(r + b_ref[...]).astype(BF16)

        @pl.when(ph == 1)
        def _l2():
            for p_ref, zs, w_ref, b_ref, es, e_out in (
                    (p1_ref, (z1a_s, z1b_s), w2a_ref, b2a_ref,
                     (e1a_s, e1b_s), e1_o),
                    (p2_ref, (z2a_s, z2b_s), w2b_ref, b2b_ref,
                     (e2a_s, e2b_s), e2_o)):
                s = p_ref[...]
                w = w_ref[...].astype(BF16)
                for i, (z_ref, e_ref) in enumerate(zip(zs, es)):
                    q = lax.dot_general(z_ref[...], w, _NN,
                                        preferred_element_type=F32)
                    r = lax.dot_general(s, q.astype(BF16), _TN,
                                        preferred_element_type=F32)
                    r = r + b_ref[...]
                    e_ref[rows, :] = r
                    if i == 0:
                        e_out[...] = r

        @pl.when(ph == 2)
        def _readout():
            readout_one(m1_ref[...], e1a_s[...], e1b_s[...],
                        e1a_s[rows, :], e1b_s[rows, :], wd1_ref[...],
                        bd1_ref[0, 0], ret1_o, reta1_o)
            readout_one(m2_ref[...], e2a_s[...], e2b_s[...],
                        e2a_s[rows, :], e2b_s[rows, :], wd2_ref[...],
                        bd2_ref[0, 0], ret2_o, reta2_o)
            x1 = e1a_s[rows, :]
            x2 = e2a_s[rows, :]
            w = aw_ref[...]
            u = au_ref[...]
            v1 = jnp.tanh(jnp.dot(x1, w, preferred_element_type=F32))
            v2 = jnp.tanh(jnp.dot(x2, w, preferred_element_type=F32))
            s1 = jnp.sum(v1 * u, axis=1, keepdims=True) + 1e-6
            s2 = jnp.sum(v2 * u, axis=1, keepdims=True) + 1e-6
            mx = jnp.maximum(s1, s2)
            q1 = jnp.exp(s1 - mx)
            q2 = jnp.exp(s2 - mx)
            den = q1 + q2
            a1 = q1 / den
            a2 = q2 / den
            alpha_o[...] = jnp.concatenate([a1, a2], axis=1)
            comb = a1 * x1 + a2 * x2
            comb_o[...] = comb
            cb = comb.astype(BF16)
            c1_s[rows, :] = lax.dot_general(
                cb, dwa_ref[...].astype(BF16), _NN,
                preferred_element_type=F32).astype(BF16)
            c2_s[rows, :] = lax.dot_general(
                cb, dwb_ref[...].astype(BF16), _NN,
                preferred_element_type=F32).astype(BF16)

        @pl.when(ph == 3)
        def _h():
            for p_ref, c_s, db_ref, h_s in (
                    (p1_ref, c1_s, dba_ref, h1_s),
                    (p2_ref, c2_s, dbb_ref, h2_s)):
                r = lax.dot_general(p_ref[...], c_s[...], _TN,
                                    preferred_element_type=F32)
                h_s[rows, :] = jax.nn.relu(r + db_ref[...]).astype(BF16)

        @pl.when(ph == 4)
        def _g():
            for p_ref, h_s, g_o in (
                    (p1_ref, h1_s, g1_o),
                    (p2_ref, h2_s, g2_o)):
                g_o[...] = lax.dot_general(
                    p_ref[...], h_s[...], _TN,
                    preferred_element_type=F32).astype(BF16)

    small = [
        (w2a, (HID, OUT)), (b2a.reshape(1, OUT), (1, OUT)),
        (wd1, (OUT, OUT)), (bd1.reshape(1, 1), (1, 1)),
        (w2b, (HID, OUT)), (b2b.reshape(1, OUT), (1, OUT)),
        (wd2, (OUT, OUT)), (bd2.reshape(1, 1), (1, 1)),
    ]
    small2 = [
        (aw, (OUT, OUT)), (au, (1, OUT)),
        (dwa, (OUT, HID)), (dba.reshape(1, HID), (1, HID)),
        (dwb, (OUT, HID)), (dbb.reshape(1, HID), (1, HID)),
        (b1a.reshape(1, HID), (1, HID)), (b1b.reshape(1, HID), (1, HID)),
    ]
    in_specs = (
        [pl.BlockSpec((N, bm), p_idx)] * 2
        + [pl.BlockSpec((N, HID), const)] * 4
        + [pl.BlockSpec(shp, const) for _, shp in small]
        + [pl.BlockSpec((bm, N), gate(2))] * 2
        + [pl.BlockSpec(shp, const) for _, shp in small2]
    )
    args = ([p1, p2, y1a, y1b, y2a, y2b]
            + [a for a, _ in small] + [mask1, mask2]
            + [a for a, _ in small2])
    out_specs = [
        pl.BlockSpec((bm, OUT), gate(1)),
        pl.BlockSpec((bm, OUT), gate(1)),
        pl.BlockSpec((bm, 2), gate(2)),
        pl.BlockSpec((bm, 2), gate(2)),
        pl.BlockSpec((bm, 2), gate(2)),
        pl.BlockSpec((bm, 2), gate(2)),
        pl.BlockSpec((bm, 2), gate(2)),
        pl.BlockSpec((bm, OUT), gate(2)),
        pl.BlockSpec((bm, HID), gate(4)),
        pl.BlockSpec((bm, HID), gate(4)),
    ]
    out_shape = [
        jax.ShapeDtypeStruct((N, OUT), F32),
        jax.ShapeDtypeStruct((N, OUT), F32),
        jax.ShapeDtypeStruct((N, 2), F32),
        jax.ShapeDtypeStruct((N, 2), F32),
        jax.ShapeDtypeStruct((N, 2), F32),
        jax.ShapeDtypeStruct((N, 2), F32),
        jax.ShapeDtypeStruct((N, 2), F32),
        jax.ShapeDtypeStruct((N, OUT), F32),
        jax.ShapeDtypeStruct((N, HID), BF16),
        jax.ShapeDtypeStruct((N, HID), BF16),
    ]
    scratch_shapes = (
        [pltpu.VMEM((N, HID), BF16)] * 4
        + [pltpu.VMEM((N, OUT), F32)] * 4
        + [pltpu.VMEM((N, HID), BF16)] * 2
        + [pltpu.VMEM((N, HID), BF16)] * 2
    )
    return pl.pallas_call(
        body,
        grid=(5, nb),
        in_specs=in_specs,
        out_specs=out_specs,
        out_shape=out_shape,
        scratch_shapes=scratch_shapes,
    )(*args)


def _ptmm2(ps, yss, biases=None, act=None, out_dtype=F32, bm=512,
           pre_ws=None):
    ng = len(ps)
    counts = [len(ys) for ys in yss]
    kcs = [ys[0].shape[1] for ys in yss]
    if pre_ws is not None:
        ncs = [w.shape[1] for w in pre_ws]
    else:
        ncs = kcs

    def body(*refs):
        refs = list(refs)
        p_refs = [refs.pop(0) for _ in range(ng)]
        y_refs = [[refs.pop(0) for _ in range(counts[g])] for g in range(ng)]
        w_refs = ([refs.pop(0) for _ in range(ng)]
                  if pre_ws is not None else [None] * ng)
        b_refs = ([refs.pop(0) for _ in range(ng)]
                  if biases is not None else [None] * ng)
        for g in range(ng):
            s = p_refs[g][...]
            if pre_ws is not None:
                w = w_refs[g][...].astype(BF16)
            for i in range(counts[g]):
                y = y_refs[g][i][...]
                if pre_ws is not None:
                    y = lax.dot_general(y, w, _NN,
                                        preferred_element_type=F32)
                    y = y.astype(BF16)
                r = lax.dot_general(s, y, _TN,
                                    preferred_element_type=F32)
                if biases is not None:
                    r = r + b_refs[g][...]
                if act is not None:
                    r = act(r)
                refs.pop(0)[...] = r.astype(out_dtype)

    in_specs = [pl.BlockSpec((N, bm), lambda m: (0, m))] * ng
    args = list(ps)
    for g in range(ng):
        in_specs += [pl.BlockSpec((N, kcs[g]), lambda m: (0, 0))] * counts[g]
        args += list(yss[g])
    if pre_ws is not None:
        for g in range(ng):
            in_specs.append(
                pl.BlockSpec((kcs[g], ncs[g]), lambda m: (0, 0)))
            args.append(pre_ws[g])
    if biases is not None:
        for g in range(ng):
            in_specs.append(pl.BlockSpec((1, ncs[g]), lambda m: (0, 0)))
            args.append(biases[g].reshape(1, ncs[g]))
    out_specs = []
    out_shape = []
    for g in range(ng):
        out_specs += [pl.BlockSpec((bm, ncs[g]), lambda m: (m, 0))] * counts[g]
        out_shape += [jax.ShapeDtypeStruct((N, ncs[g]), out_dtype)] * counts[g]
    flat = pl.pallas_call(
        body,
        grid=(N // bm,),
        in_specs=in_specs,
        out_specs=out_specs,
        out_shape=out_shape,
    )(*args)
    outs = []
    k = 0
    for g in range(ng):
        outs.append(list(flat[k:k + counts[g]]))
        k += counts[g]
    return outs


def _mm(avs, bmat, bias=None, act=None, out_dtype=F32, bm=512, bn=512):
    nd = len(avs)
    m_, k_ = avs[0].shape
    nc = bmat.shape[1]
    bn = min(bn, -(-nc // 128) * 128)
    grid = (m_ // bm, -(-nc // bn))
    k_main = (k_ // 128) * 128
    ragged = k_main != k_
    ktail_blk = k_main // 128

    def body(*refs):
        refs = list(refs)
        a_refs = [refs.pop(0) for _ in range(nd)]
        if ragged:
            at_refs = [refs.pop(0) for _ in range(nd)]
        b_ref = refs.pop(0)
        if ragged:
            bt_ref = refs.pop(0)
        if bias is not None:
            bias_ref = refs.pop(0)
        o_refs = refs
        bmain = b_ref[...].astype(BF16)
        if ragged:
            kid = lax.broadcasted_iota(jnp.int32, (128, bn), 0)
            btail = jnp.where(kid < (k_ - k_main), bt_ref[...], 0.0)
            btail = btail.astype(BF16)
        for i in range(nd):
            r = lax.dot_general(a_refs[i][...].astype(BF16), bmain, _NN,
                                preferred_element_type=F32)
            if ragged:
                kida = lax.broadcasted_iota(jnp.int32, (bm, 128), 1)
                atail = jnp.where(kida < (k_ - k_main), at_refs[i][...], 0.0)
                r = r + lax.dot_general(atail.astype(BF16), btail, _NN,
                                        preferred_element_type=F32)
            if bias is not None:
                r = r + bias_ref[...]
            if act is not None:
                r = act(r)
            o_refs[i][...] = r.astype(out_dtype)

    in_specs = [pl.BlockSpec((bm, k_main), lambda m, n: (m, 0))] * nd
    args = list(avs)
    if ragged:
        in_specs += [pl.BlockSpec((bm, 128),
                                  lambda m, n: (m, ktail_blk))] * nd
        args += list(avs)
    in_specs.append(pl.BlockSpec((k_main, bn), lambda m, n: (0, n)))
    args.append(bmat)
    if ragged:
        in_specs.append(pl.BlockSpec((128, bn), lambda m, n: (ktail_blk, n)))
        args.append(bmat)
    if bias is not None:
        in_specs.append(pl.BlockSpec((1, bn), lambda m, n: (0, n)))
        args.append(bias)
    return pl.pallas_call(
        body,
        grid=grid,
        in_specs=in_specs,
        out_specs=[pl.BlockSpec((bm, bn), lambda m, n: (m, n))] * nd,
        out_shape=[jax.ShapeDtypeStruct((m_, nc), out_dtype)] * nd,
        compiler_params=_PARAMS,
    )(*args)


def _readout_disc_attn(groups, aw, u_row, dec_w1s):
    bmr = 512
    nh = dec_w1s[0].shape[1]

    def one(mk, ef, eaf, er, ear, w, bdv, ret_ref, reta_ref):
        rs = jnp.sum(mk, axis=1, keepdims=True)
        mb = mk.astype(BF16)
        vs1 = lax.dot_general(mb, ef.astype(BF16), _NN,
                              preferred_element_type=F32)
        vs2 = lax.dot_general(mb, eaf.astype(BF16), _NN,
                              preferred_element_type=F32)
        gp = vs1 / rs
        gap = vs2 / rs
        n1 = jnp.maximum(jnp.sqrt(jnp.sum(gp * gp, axis=1, keepdims=True)),
                         1e-12)
        n2 = jnp.maximum(jnp.sqrt(jnp.sum(gap * gap, axis=1, keepdims=True)),
                         1e-12)
        g = jax.nn.sigmoid(gp / n1)
        ga = jax.nn.sigmoid(gap / n2)
        u = jnp.dot(er, w, preferred_element_type=F32)
        ua = jnp.dot(ear, w, preferred_element_type=F32)
        ret_ref[...] = jnp.concatenate([
            jnp.sum(u * g, axis=1, keepdims=True) + bdv,
            jnp.sum(ua * g, axis=1, keepdims=True) + bdv], axis=1)
        reta_ref[...] = jnp.concatenate([
            jnp.sum(ua * ga, axis=1, keepdims=True) + bdv,
            jnp.sum(u * ga, axis=1, keepdims=True) + bdv], axis=1)

    def body(m1, e1f, e1af, e1r, e1ar, w1, b1,
             m2, e2f, e2af, e2r, e2ar, w2, b2,
             aw_ref, u_ref, dw1_ref, dw2_ref,
             ret1, reta1, ret2, reta2, alpha_ref, comb_ref, c1_ref, c2_ref):
        one(m1[...], e1f[...], e1af[...], e1r[...], e1ar[...], w1[...],
            b1[0, 0], ret1, reta1)
        one(m2[...], e2f[...], e2af[...], e2r[...], e2ar[...], w2[...],
            b2[0, 0], ret2, reta2)
        x1 = e1r[...]
        x2 = e2r[...]
        w = aw_ref[...]
        u = u_ref[...]
        v1 = jnp.tanh(jnp.dot(x1, w, preferred_element_type=F32))
        v2 = jnp.tanh(jnp.dot(x2, w, preferred_element_type=F32))
        s1 = jnp.sum(v1 * u, axis=1, keepdims=True) + 1e-6
        s2 = jnp.sum(v2 * u, axis=1, keepdims=True) + 1e-6
        mx = jnp.maximum(s1, s2)
        p1 = jnp.exp(s1 - mx)
        p2 = jnp.exp(s2 - mx)
        den = p1 + p2
        a1 = p1 / den
        a2 = p2 / den
        alpha_ref[...] = jnp.concatenate([a1, a2], axis=1)
        comb = a1 * x1 + a2 * x2
        comb_ref[...] = comb
        cb = comb.astype(BF16)
        c1_ref[...] = lax.dot_general(
            cb, dw1_ref[...].astype(BF16), _NN,
            preferred_element_type=F32).astype(BF16)
        c2_ref[...] = lax.dot_general(
            cb, dw2_ref[...].astype(BF16), _NN,
            preferred_element_type=F32).astype(BF16)

    specs_one = [
        pl.BlockSpec((bmr, N), lambda m: (m, 0)),
        pl.BlockSpec((N, OUT), lambda m: (0, 0)),
        pl.BlockSpec((N, OUT), lambda m: (0, 0)),
        pl.BlockSpec((bmr, OUT), lambda m: (m, 0)),
        pl.BlockSpec((bmr, OUT), lambda m: (m, 0)),
        pl.BlockSpec((OUT, OUT), lambda m: (0, 0)),
        pl.BlockSpec((1, 1), lambda m: (0, 0)),
    ]
    args = []
    for mask, ec, es, wd, bd in groups:
        args += [mask, ec, es, ec, es, wd, bd.reshape(1, 1)]
    args += [aw, u_row, dec_w1s[0], dec_w1s[1]]
    return pl.pallas_call(
        body,
        grid=(N // bmr,),
        in_specs=specs_one + specs_one + [
            pl.BlockSpec((OUT, OUT), lambda m: (0, 0)),
            pl.BlockSpec((1, OUT), lambda m: (0, 0)),
            pl.BlockSpec((OUT, nh), lambda m: (0, 0)),
            pl.BlockSpec((OUT, nh), lambda m: (0, 0)),
        ],
        out_specs=[pl.BlockSpec((bmr, 2), lambda m: (m, 0))] * 4 + [
            pl.BlockSpec((bmr, 2), lambda m: (m, 0)),
            pl.BlockSpec((bmr, OUT), lambda m: (m, 0)),
            pl.BlockSpec((bmr, nh), lambda m: (m, 0)),
            pl.BlockSpec((bmr, nh), lambda m: (m, 0)),
        ],
        out_shape=[jax.ShapeDtypeStruct((N, 2), F32)] * 4 + [
            jax.ShapeDtypeStruct((N, 2), F32),
            jax.ShapeDtypeStruct((N, OUT), F32),
            jax.ShapeDtypeStruct((N, nh), BF16),
            jax.ShapeDtypeStruct((N, nh), BF16),
        ],
    )(*args)


def kernel(omics1_feat_shuffle, omics2_feat_shuffle, omics1_feat, omics2_feat,
           omics1_adj, omics2_adj, omics1_graph_neigh, omics2_graph_neigh,
           o1_enc_W1, o1_enc_b1, o1_enc_W2, o1_enc_b2, o1_disc_W, o1_disc_b,
           o2_enc_W1, o2_enc_b1, o2_enc_W2, o2_enc_b2, o2_disc_W, o2_disc_b,
           o1_dec_W1, o1_dec_b1, o1_dec_W2, o1_dec_b2,
           o2_dec_W1, o2_dec_b1, o2_dec_W2, o2_dec_b2, att_w, att_u):
    d1, d2 = _dinv2(omics1_adj, omics2_adj)
    p1m, p2m = _norm2(omics1_adj, d1, omics2_adj, d2)

    y1a, y1b = _mm([omics1_feat, omics1_feat_shuffle], o1_enc_W1,
                   out_dtype=BF16)
    y2a, y2b = _mm([omics2_feat, omics2_feat_shuffle], o2_enc_W1,
                   out_dtype=BF16)

    (e1a, e2a, o1_ret, o1_ret_a, o2_ret, o2_ret_a,
     alpha, comb, g1, g2) = _core(
        p1m, p2m, y1a, y1b, y2a, y2b,
        o1_enc_W2, o1_enc_b2, o1_disc_W, o1_disc_b,
        o2_enc_W2, o2_enc_b2, o2_disc_W, o2_disc_b,
        omics1_graph_neigh, omics2_graph_neigh,
        att_w, att_u.reshape(1, OUT),
        o1_dec_W1, o1_dec_b1, o2_dec_W1, o2_dec_b1,
        o1_enc_b1, o2_enc_b1)

    (o1_rec,) = _mm([g1], o1_dec_W2, bias=o1_dec_b2.reshape(1, -1))
    (o2_rec,) = _mm([g2], o2_dec_W2, bias=o2_dec_b2.reshape(1, -1))

    return (e1a, o1_rec, o1_ret, o1_ret_a,
            e2a, o2_rec, o2_ret, o2_ret_a, comb, alpha)


# speedup vs baseline: 1.2598x; 1.0122x over previous
import jax
import jax.numpy as jnp
from jax import lax
from jax.experimental import pallas as pl
from jax.experimental.pallas import tpu as pltpu

N = 2048
HID = 256
OUT = 128
F32 = jnp.float32
BF16 = jnp.bfloat16

_TN = (((0,), (0,)), ((), ()))
_NN = (((1,), (0,)), ((), ()))
_PARAMS = pltpu.CompilerParams(
    dimension_semantics=("parallel", "parallel"))


def _dinv2(adj1, adj2):
    bj = 512

    def body(a1_ref, a2_ref, o1_ref, o2_ref):
        j = pl.program_id(0)
        for a_ref, o_ref in ((a1_ref, o1_ref), (a2_ref, o2_ref)):
            t = a_ref[...]
            b = jnp.where(t != 0, 1.0, 0.0)
            rid = lax.broadcasted_iota(jnp.int32, t.shape, 0)
            cid = j * bj + lax.broadcasted_iota(jnp.int32, t.shape, 1)
            d = jnp.where(rid == cid, 1.0, b)
            o_ref[...] = 1.0 / jnp.sqrt(jnp.sum(d, axis=0))

    return pl.pallas_call(
        body,
        grid=(N // bj,),
        in_specs=[pl.BlockSpec((N, bj), lambda j: (0, j))] * 2,
        out_specs=[pl.BlockSpec((bj,), lambda j: (j,))] * 2,
        out_shape=[jax.ShapeDtypeStruct((N,), F32)] * 2,
    )(adj1, adj2)


def _norm2(adj1, d1, adj2, d2):
    bj = 512

    def body(a1_ref, d1c_ref, d1r_ref, a2_ref, d2c_ref, d2r_ref,
             o1_ref, o2_ref):
        j = pl.program_id(0)
        for a_ref, dc_ref, dr_ref, o_ref in (
                (a1_ref, d1c_ref, d1r_ref, o1_ref),
                (a2_ref, d2c_ref, d2r_ref, o2_ref)):
            t = a_ref[...]
            b = jnp.where(t != 0, 1.0, 0.0)
            rid = lax.broadcasted_iota(jnp.int32, t.shape, 0)
            cid = j * bj + lax.broadcasted_iota(jnp.int32, t.shape, 1)
            d = jnp.where(rid == cid, 1.0, b)
            o_ref[...] = ((dc_ref[...] * d) * dr_ref[...]).astype(BF16)

    specs_one = [
        pl.BlockSpec((N, bj), lambda j: (0, j)),
        pl.BlockSpec((N, 1), lambda j: (0, 0)),
        pl.BlockSpec((1, bj), lambda j: (0, j)),
    ]
    return pl.pallas_call(
        body,
        grid=(N // bj,),
        in_specs=specs_one + specs_one,
        out_specs=[pl.BlockSpec((N, bj), lambda j: (0, j))] * 2,
        out_shape=[jax.ShapeDtypeStruct((N, N), BF16)] * 2,
    )(adj1, d1.reshape(N, 1), d1.reshape(1, N),
      adj2, d2.reshape(N, 1), d2.reshape(1, N))


def _core(p1, p2, y1a, y1b, y2a, y2b,
          w2a, b2a, wd1, bd1, w2b, b2b, wd2, bd2,
          mask1, mask2, aw, au, dwa, dba, dwb, dbb,
          b1a, b1b):
    bm = 512
    nb = N // bm

    def gate(phx):
        return lambda ph, m: (jnp.where(ph < phx, 0,
                                        jnp.where(ph == phx, m, nb - 1)), 0)

    def const(ph, m):
        return (0, 0)

    p_idx = lambda ph, m: (0, jnp.where(ph == 0, m, nb - 1))

    def readout_one(mk, ef, eaf_bf, er, ear_bf, w, bdv, ret_ref, reta_ref):
        rs = jnp.sum(mk, axis=1, keepdims=True)
        mb = mk.astype(BF16)
        vs1 = lax.dot_general(mb, ef.astype(BF16), _NN,
                              preferred_element_type=F32)
        vs2 = lax.dot_general(mb, eaf_bf, _NN,
                              preferred_element_type=F32)
        ear = ear_bf.astype(F32)
        gp = vs1 / rs
        gap = vs2 / rs
        n1 = jnp.maximum(jnp.sqrt(jnp.sum(gp * gp, axis=1, keepdims=True)),
                         1e-12)
        n2 = jnp.maximum(jnp.sqrt(jnp.sum(gap * gap, axis=1, keepdims=True)),
                         1e-12)
        g = jax.nn.sigmoid(gp / n1)
        ga = jax.nn.sigmoid(gap / n2)
        u = jnp.dot(er, w, preferred_element_type=F32)
        ua = jnp.dot(ear, w, preferred_element_type=F32)
        ret_ref[...] = jnp.concatenate([
            jnp.sum(u * g, axis=1, keepdims=True) + bdv,
            jnp.sum(ua * g, axis=1, keepdims=True) + bdv], axis=1)
        reta_ref[...] = jnp.concatenate([
            jnp.sum(ua * ga, axis=1, keepdims=True) + bdv,
            jnp.sum(u * ga, axis=1, keepdims=True) + bdv], axis=1)

    def body(p1_ref, p2_ref, y1a_ref, y1b_ref, y2a_ref, y2b_ref,
             w2a_ref, b2a_ref, wd1_ref, bd1_ref,
             w2b_ref, b2b_ref, wd2_ref, bd2_ref,
             m1_ref, m2_ref, aw_ref, au_ref,
             dwa_ref, dba_ref, dwb_ref, dbb_ref, b1a_ref, b1b_ref,
             e1_o, e2_o, ret1_o, reta1_o, ret2_o, reta2_o,
             alpha_o, comb_o, g1_o, g2_o,
             p1_s, p2_s,
             z1a_s, z1b_s, z2a_s, z2b_s,
             e1a_s, e1b_s, e2a_s, e2b_s):
        ph = pl.program_id(0)
        m = pl.program_id(1)
        rows = pl.ds(m * bm, bm)
        c1_s, c2_s, h1_s, h2_s = z1a_s, z1b_s, z2a_s, z2b_s

        @pl.when(ph == 0)
        def _l1():
            for p_ref, p_s, ys, b_ref, zs in (
                    (p1_ref, p1_s, (y1a_ref, y1b_ref), b1a_ref,
                     (z1a_s, z1b_s)),
                    (p2_ref, p2_s, (y2a_ref, y2b_ref), b1b_ref,
                     (z2a_s, z2b_s))):
                s = p_ref[...]
                p_s[m] = s
                for y_ref, z_ref in zip(ys, zs):
                    r = lax.dot_general(s, y_ref[...], _TN,
                                        preferred_element_type=F32)
                    z_ref[rows, :] = jax.nn.relu(r + b_ref[...]).astype(BF16)

        @pl.when(ph == 1)
        def _l2():
            for p_s, zs, w_ref, b_ref, es, e_out in (
                    (p1_s, (z1a_s, z1b_s), w2a_ref, b2a_ref,
                     (e1a_s, e1b_s), e1_o),
                    (p2_s, (z2a_s, z2b_s), w2b_ref, b2b_ref,
                     (e2a_s, e2b_s), e2_o)):
                s = p_s[m]
                w = w_ref[...].astype(BF16)
                for i, (z_ref, e_ref) in enumerate(zip(zs, es)):
                    q = lax.dot_general(z_ref[...], w, _NN,
                                        preferred_element_type=F32)
                    r = lax.dot_general(s, q.astype(BF16), _TN,
                                        preferred_element_type=F32)
                    r = r + b_ref[...]
                    if i == 0:
                        e_ref[rows, :] = r
                        e_out[...] = r
                    else:
                        e_ref[rows, :] = r.astype(BF16)

        @pl.when(ph == 2)
        def _readout():
            readout_one(m1_ref[...], e1a_s[...], e1b_s[...],
                        e1a_s[rows, :], e1b_s[rows, :], wd1_ref[...],
                        bd1_ref[0, 0], ret1_o, reta1_o)
            readout_one(m2_ref[...], e2a_s[...], e2b_s[...],
                        e2a_s[rows, :], e2b_s[rows, :], wd2_ref[...],
                        bd2_ref[0, 0], ret2_o, reta2_o)
            x1 = e1a_s[rows, :]
            x2 = e2a_s[rows, :]
            w = aw_ref[...]
            u = au_ref[...]
            v1 = jnp.tanh(jnp.dot(x1, w, preferred_element_type=F32))
            v2 = jnp.tanh(jnp.dot(x2, w, preferred_element_type=F32))
            s1 = jnp.sum(v1 * u, axis=1, keepdims=True) + 1e-6
            s2 = jnp.sum(v2 * u, axis=1, keepdims=True) + 1e-6
            mx = jnp.maximum(s1, s2)
            q1 = jnp.exp(s1 - mx)
            q2 = jnp.exp(s2 - mx)
            den = q1 + q2
            a1 = q1 / den
            a2 = q2 / den
            alpha_o[...] = jnp.concatenate([a1, a2], axis=1)
            comb = a1 * x1 + a2 * x2
            comb_o[...] = comb
            cb = comb.astype(BF16)
            c1_s[rows, :] = lax.dot_general(
                cb, dwa_ref[...].astype(BF16), _NN,
                preferred_element_type=F32).astype(BF16)
            c2_s[rows, :] = lax.dot_general(
                cb, dwb_ref[...].astype(BF16), _NN,
                preferred_element_type=F32).astype(BF16)

        @pl.when(ph == 3)
        def _h():
            for p_s, c_s, db_ref, h_s in (
                    (p1_s, c1_s, dba_ref, h1_s),
                    (p2_s, c2_s, dbb_ref, h2_s)):
                r = lax.dot_general(p_s[m], c_s[...], _TN,
                                    preferred_element_type=F32)
                h_s[rows, :] = jax.nn.relu(r + db_ref[...]).astype(BF16)

        @pl.when(ph == 4)
        def _g():
            for p_s, h_s, g_o in (
                    (p1_s, h1_s, g1_o),
                    (p2_s, h2_s, g2_o)):
                g_o[...] = lax.dot_general(
                    p_s[m], h_s[...], _TN,
                    preferred_element_type=F32).astype(BF16)

    small = [
        (w2a, (HID, OUT)), (b2a.reshape(1, OUT), (1, OUT)),
        (wd1, (OUT, OUT)), (bd1.reshape(1, 1), (1, 1)),
        (w2b, (HID, OUT)), (b2b.reshape(1, OUT), (1, OUT)),
        (wd2, (OUT, OUT)), (bd2.reshape(1, 1), (1, 1)),
    ]
    small2 = [
        (aw, (OUT, OUT)), (au, (1, OUT)),
        (dwa, (OUT, HID)), (dba.reshape(1, HID), (1, HID)),
        (dwb, (OUT, HID)), (dbb.reshape(1, HID), (1, HID)),
        (b1a.reshape(1, HID), (1, HID)), (b1b.reshape(1, HID), (1, HID)),
    ]
    in_specs = (
        [pl.BlockSpec((N, bm), p_idx)] * 2
        + [pl.BlockSpec((N, HID), const)] * 4
        + [pl.BlockSpec(shp, const) for _, shp in small]
        + [pl.BlockSpec((bm, N), gate(2))] * 2
        + [pl.BlockSpec(shp, const) for _, shp in small2]
    )
    args = ([p1, p2, y1a, y1b, y2a, y2b]
            + [a for a, _ in small] + [mask1, mask2]
            + [a for a, _ in small2])
    out_specs = [
        pl.BlockSpec((bm, OUT), gate(1)),
        pl.BlockSpec((bm, OUT), gate(1)),
        pl.BlockSpec((bm, 2), gate(2)),
        pl.BlockSpec((bm, 2), gate(2)),
        pl.BlockSpec((bm, 2), gate(2)),
        pl.BlockSpec((bm, 2), gate(2)),
        pl.BlockSpec((bm, 2), gate(2)),
        pl.BlockSpec((bm, OUT), gate(2)),
        pl.BlockSpec((bm, HID), gate(4)),
        pl.BlockSpec((bm, HID), gate(4)),
    ]
    out_shape = [
        jax.ShapeDtypeStruct((N, OUT), F32),
        jax.ShapeDtypeStruct((N, OUT), F32),
        jax.ShapeDtypeStruct((N, 2), F32),
        jax.ShapeDtypeStruct((N, 2), F32),
        jax.ShapeDtypeStruct((N, 2), F32),
        jax.ShapeDtypeStruct((N, 2), F32),
        jax.ShapeDtypeStruct((N, 2), F32),
        jax.ShapeDtypeStruct((N, OUT), F32),
        jax.ShapeDtypeStruct((N, HID), BF16),
        jax.ShapeDtypeStruct((N, HID), BF16),
    ]
    scratch_shapes = (
        [pltpu.VMEM((nb, N, bm), BF16)] * 2
        + [pltpu.VMEM((N, HID), BF16)] * 4
        + [pltpu.VMEM((N, OUT), F32),
           pltpu.VMEM((N, OUT), BF16),
           pltpu.VMEM((N, OUT), F32),
           pltpu.VMEM((N, OUT), BF16)]
    )
    return pl.pallas_call(
        body,
        grid=(5, nb),
        in_specs=in_specs,
        out_specs=out_specs,
        out_shape=out_shape,
        scratch_shapes=scratch_shapes,
    )(*args)


def _ptmm2(ps, yss, biases=None, act=None, out_dtype=F32, bm=512,
           pre_ws=None):
    ng = len(ps)
    counts = [len(ys) for ys in yss]
    kcs = [ys[0].shape[1] for ys in yss]
    if pre_ws is not None:
        ncs = [w.shape[1] for w in pre_ws]
    else:
        ncs = kcs

    def body(*refs):
        refs = list(refs)
        p_refs = [refs.pop(0) for _ in range(ng)]
        y_refs = [[refs.pop(0) for _ in range(counts[g])] for g in range(ng)]
        w_refs = ([refs.pop(0) for _ in range(ng)]
                  if pre_ws is not None else [None] * ng)
        b_refs = ([refs.pop(0) for _ in range(ng)]
                  if biases is not None else [None] * ng)
        for g in range(ng):
            s = p_refs[g][...]
            if pre_ws is not None:
                w = w_refs[g][...].astype(BF16)
            for i in range(counts[g]):
                y = y_refs[g][i][...]
                if pre_ws is not None:
                    y = lax.dot_general(y, w, _NN,
                                        preferred_element_type=F32)
                    y = y.astype(BF16)
                r = lax.dot_general(s, y, _TN,
                                    preferred_element_type=F32)
                if biases is not None:
                    r = r + b_refs[g][...]
                if act is not None:
                    r = act(r)
                refs.pop(0)[...] = r.astype(out_dtype)

    in_specs = [pl.BlockSpec((N, bm), lambda m: (0, m))] * ng
    args = list(ps)
    for g in range(ng):
        in_specs += [pl.BlockSpec((N, kcs[g]), lambda m: (0, 0))] * counts[g]
        args += list(yss[g])
    if pre_ws is not None:
        for g in range(ng):
            in_specs.append(
                pl.BlockSpec((kcs[g], ncs[g]), lambda m: (0, 0)))
            args.append(pre_ws[g])
    if biases is not None:
        for g in range(ng):
            in_specs.append(pl.BlockSpec((1, ncs[g]), lambda m: (0, 0)))
            args.append(biases[g].reshape(1, ncs[g]))
    out_specs = []
    out_shape = []
    for g in range(ng):
        out_specs += [pl.BlockSpec((bm, ncs[g]), lambda m: (m, 0))] * counts[g]
        out_shape += [jax.ShapeDtypeStruct((N, ncs[g]), out_dtype)] * counts[g]
    flat = pl.pallas_call(
        body,
        grid=(N // bm,),
        in_specs=in_specs,
        out_specs=out_specs,
        out_shape=out_shape,
    )(*args)
    outs = []
    k = 0
    for g in range(ng):
        outs.append(list(flat[k:k + counts[g]]))
        k += counts[g]
    return outs


def _mm(avs, bmat, bias=None, act=None, out_dtype=F32, bm=512, bn=512):
    nd = len(avs)
    m_, k_ = avs[0].shape
    nc = bmat.shape[1]
    bn = min(bn, -(-nc // 128) * 128)
    grid = (m_ // bm, -(-nc // bn))
    k_main = (k_ // 128) * 128
    ragged = k_main != k_
    ktail_blk = k_main // 128

    def body(*refs):
        refs = list(refs)
        a_refs = [refs.pop(0) for _ in range(nd)]
        if ragged:
            at_refs = [refs.pop(0) for _ in range(nd)]
        b_ref = refs.pop(0)
        if ragged:
            bt_ref = refs.pop(0)
        if bias is not None:
            bias_ref = refs.pop(0)
        o_refs = refs
        bmain = b_ref[...].astype(BF16)
        if ragged:
            kid = lax.broadcasted_iota(jnp.int32, (128, bn), 0)
            btail = jnp.where(kid < (k_ - k_main), bt_ref[...], 0.0)
            btail = btail.astype(BF16)
        for i in range(nd):
            r = lax.dot_general(a_refs[i][...].astype(BF16), bmain, _NN,
                                preferred_element_type=F32)
            if ragged:
                kida = lax.broadcasted_iota(jnp.int32, (bm, 128), 1)
                atail = jnp.where(kida < (k_ - k_main), at_refs[i][...], 0.0)
                r = r + lax.dot_general(atail.astype(BF16), btail, _NN,
                                        preferred_element_type=F32)
            if bias is not None:
                r = r + bias_ref[...]
            if act is not None:
                r = act(r)
            o_refs[i][...] = r.astype(out_dtype)

    in_specs = [pl.BlockSpec((bm, k_main), lambda m, n: (m, 0))] * nd
    args = list(avs)
    if ragged:
        in_specs += [pl.BlockSpec((bm, 128),
                                  lambda m, n: (m, ktail_blk))] * nd
        args += list(avs)
    in_specs.append(pl.BlockSpec((k_main, bn), lambda m, n: (0, n)))
    args.append(bmat)
    if ragged:
        in_specs.append(pl.BlockSpec((128, bn), lambda m, n: (ktail_blk, n)))
        args.append(bmat)
    if bias is not None:
        in_specs.append(pl.BlockSpec((1, bn), lambda m, n: (0, n)))
        args.append(bias)
    return pl.pallas_call(
        body,
        grid=grid,
        in_specs=in_specs,
        out_specs=[pl.BlockSpec((bm, bn), lambda m, n: (m, n))] * nd,
        out_shape=[jax.ShapeDtypeStruct((m_, nc), out_dtype)] * nd,
        compiler_params=_PARAMS,
    )(*args)


def _readout_disc_attn(groups, aw, u_row, dec_w1s):
    bmr = 512
    nh = dec_w1s[0].shape[1]

    def one(mk, ef, eaf, er, ear, w, bdv, ret_ref, reta_ref):
        rs = jnp.sum(mk, axis=1, keepdims=True)
        mb = mk.astype(BF16)
        vs1 = lax.dot_general(mb, ef.astype(BF16), _NN,
                              preferred_element_type=F32)
        vs2 = lax.dot_general(mb, eaf.astype(BF16), _NN,
                              preferred_element_type=F32)
        gp = vs1 / rs
        gap = vs2 / rs
        n1 = jnp.maximum(jnp.sqrt(jnp.sum(gp * gp, axis=1, keepdims=True)),
                         1e-12)
        n2 = jnp.maximum(jnp.sqrt(jnp.sum(gap * gap, axis=1, keepdims=True)),
                         1e-12)
        g = jax.nn.sigmoid(gp / n1)
        ga = jax.nn.sigmoid(gap / n2)
        u = jnp.dot(er, w, preferred_element_type=F32)
        ua = jnp.dot(ear, w, preferred_element_type=F32)
        ret_ref[...] = jnp.concatenate([
            jnp.sum(u * g, axis=1, keepdims=True) + bdv,
            jnp.sum(ua * g, axis=1, keepdims=True) + bdv], axis=1)
        reta_ref[...] = jnp.concatenate([
            jnp.sum(ua * ga, axis=1, keepdims=True) + bdv,
            jnp.sum(u * ga, axis=1, keepdims=True) + bdv], axis=1)

    def body(m1, e1f, e1af, e1r, e1ar, w1, b1,
             m2, e2f, e2af, e2r, e2ar, w2, b2,
             aw_ref, u_ref, dw1_ref, dw2_ref,
             ret1, reta1, ret2, reta2, alpha_ref, comb_ref, c1_ref, c2_ref):
        one(m1[...], e1f[...], e1af[...], e1r[...], e1ar[...], w1[...],
            b1[0, 0], ret1, reta1)
        one(m2[...], e2f[...], e2af[...], e2r[...], e2ar[...], w2[...],
            b2[0, 0], ret2, reta2)
        x1 = e1r[...]
        x2 = e2r[...]
        w = aw_ref[...]
        u = u_ref[...]
        v1 = jnp.tanh(jnp.dot(x1, w, preferred_element_type=F32))
        v2 = jnp.tanh(jnp.dot(x2, w, preferred_element_type=F32))
        s1 = jnp.sum(v1 * u, axis=1, keepdims=True) + 1e-6
        s2 = jnp.sum(v2 * u, axis=1, keepdims=True) + 1e-6
        mx = jnp.maximum(s1, s2)
        p1 = jnp.exp(s1 - mx)
        p2 = jnp.exp(s2 - mx)
        den = p1 + p2
        a1 = p1 / den
        a2 = p2 / den
        alpha_ref[...] = jnp.concatenate([a1, a2], axis=1)
        comb = a1 * x1 + a2 * x2
        comb_ref[...] = comb
        cb = comb.astype(BF16)
        c1_ref[...] = lax.dot_general(
            cb, dw1_ref[...].astype(BF16), _NN,
            preferred_element_type=F32).astype(BF16)
        c2_ref[...] = lax.dot_general(
            cb, dw2_ref[...].astype(BF16), _NN,
            preferred_element_type=F32).astype(BF16)

    specs_one = [
        pl.BlockSpec((bmr, N), lambda m: (m, 0)),
        pl.BlockSpec((N, OUT), lambda m: (0, 0)),
        pl.BlockSpec((N, OUT), lambda m: (0, 0)),
        pl.BlockSpec((bmr, OUT), lambda m: (m, 0)),
        pl.BlockSpec((bmr, OUT), lambda m: (m, 0)),
        pl.BlockSpec((OUT, OUT), lambda m: (0, 0)),
        pl.BlockSpec((1, 1), lambda m: (0, 0)),
    ]
    args = []
    for mask, ec, es, wd, bd in groups:
        args += [mask, ec, es, ec, es, wd, bd.reshape(1, 1)]
    args += [aw, u_row, dec_w1s[0], dec_w1s[1]]
    return pl.pallas_call(
        body,
        grid=(N // bmr,),
        in_specs=specs_one + specs_one + [
            pl.BlockSpec((OUT, OUT), lambda m: (0, 0)),
            pl.BlockSpec((1, OUT), lambda m: (0, 0)),
            pl.BlockSpec((OUT, nh), lambda m: (0, 0)),
            pl.BlockSpec((OUT, nh), lambda m: (0, 0)),
        ],
        out_specs=[pl.BlockSpec((bmr, 2), lambda m: (m, 0))] * 4 + [
            pl.BlockSpec((bmr, 2), lambda m: (m, 0)),
            pl.BlockSpec((bmr, OUT), lambda m: (m, 0)),
            pl.BlockSpec((bmr, nh), lambda m: (m, 0)),
            pl.BlockSpec((bmr, nh), lambda m: (m, 0)),
        ],
        out_shape=[jax.ShapeDtypeStruct((N, 2), F32)] * 4 + [
            jax.ShapeDtypeStruct((N, 2), F32),
            jax.ShapeDtypeStruct((N, OUT), F32),
            jax.ShapeDtypeStruct((N, nh), BF16),
            jax.ShapeDtypeStruct((N, nh), BF16),
        ],
    )(*args)


def kernel(omics1_feat_shuffle, omics2_feat_shuffle, omics1_feat, omics2_feat,
           omics1_adj, omics2_adj, omics1_graph_neigh, omics2_graph_neigh,
           o1_enc_W1, o1_enc_b1, o1_enc_W2, o1_enc_b2, o1_disc_W, o1_disc_b,
           o2_enc_W1, o2_enc_b1, o2_enc_W2, o2_enc_b2, o2_disc_W, o2_disc_b,
           o1_dec_W1, o1_dec_b1, o1_dec_W2, o1_dec_b2,
           o2_dec_W1, o2_dec_b1, o2_dec_W2, o2_dec_b2, att_w, att_u):
    d1, d2 = _dinv2(omics1_adj, omics2_adj)
    p1m, p2m = _norm2(omics1_adj, d1, omics2_adj, d2)

    y1a, y1b = _mm([omics1_feat, omics1_feat_shuffle], o1_enc_W1,
                   out_dtype=BF16)
    y2a, y2b = _mm([omics2_feat, omics2_feat_shuffle], o2_enc_W1,
                   out_dtype=BF16)

    (e1a, e2a, o1_ret, o1_ret_a, o2_ret, o2_ret_a,
     alpha, comb, g1, g2) = _core(
        p1m, p2m, y1a, y1b, y2a, y2b,
        o1_enc_W2, o1_enc_b2, o1_disc_W, o1_disc_b,
        o2_enc_W2, o2_enc_b2, o2_disc_W, o2_disc_b,
        omics1_graph_neigh, omics2_graph_neigh,
        att_w, att_u.reshape(1, OUT),
        o1_dec_W1, o1_dec_b1, o2_dec_W1, o2_dec_b1,
        o1_enc_b1, o2_enc_b1)

    (o1_rec,) = _mm([g1], o1_dec_W2, bias=o1_dec_b2.reshape(1, -1))
    (o2_rec,) = _mm([g2], o2_dec_W2, bias=o2_dec_b2.reshape(1, -1))

    return (e1a, o1_rec, o1_ret, o1_ret_a,
            e2a, o2_rec, o2_ret, o2_ret_a, comb, alpha)


# speedup vs baseline: 1.2627x; 1.0023x over previous
import jax
import jax.numpy as jnp
from jax import lax
from jax.experimental import pallas as pl
from jax.experimental.pallas import tpu as pltpu

N = 2048
HID = 256
OUT = 128
F32 = jnp.float32
BF16 = jnp.bfloat16

_TN = (((0,), (0,)), ((), ()))
_NN = (((1,), (0,)), ((), ()))
_PARAMS = pltpu.CompilerParams(
    dimension_semantics=("parallel", "parallel"))


def _core(adj1, adj2, y1a, y1b, y2a, y2b,
          w2a, b2a, wd1, bd1, w2b, b2b, wd2, bd2,
          mask1, mask2, aw, au, dwa, dba, dwb, dbb,
          b1a, b1b):
    bm = 256
    nb = N // bm

    def gate(phx):
        return lambda ph, m: (jnp.where(ph < phx, 0,
                                        jnp.where(ph == phx, m, nb - 1)), 0)

    def const(ph, m):
        return (0, 0)

    p_idx = lambda ph, m: (0, jnp.where(ph == 0, m, nb - 1))

    def readout_one(mk, ef, eaf_bf, er, ear_bf, w, bdv, ret_ref, reta_ref):
        rs = jnp.sum(mk, axis=1, keepdims=True)
        mb = mk.astype(BF16)
        vs1 = lax.dot_general(mb, ef.astype(BF16), _NN,
                              preferred_element_type=F32)
        vs2 = lax.dot_general(mb, eaf_bf, _NN,
                              preferred_element_type=F32)
        ear = ear_bf.astype(F32)
        gp = vs1 / rs
        gap = vs2 / rs
        n1 = jnp.maximum(jnp.sqrt(jnp.sum(gp * gp, axis=1, keepdims=True)),
                         1e-12)
        n2 = jnp.maximum(jnp.sqrt(jnp.sum(gap * gap, axis=1, keepdims=True)),
                         1e-12)
        g = jax.nn.sigmoid(gp / n1)
        ga = jax.nn.sigmoid(gap / n2)
        u = jnp.dot(er, w, preferred_element_type=F32)
        ua = jnp.dot(ear, w, preferred_element_type=F32)
        ret_ref[...] = jnp.concatenate([
            jnp.sum(u * g, axis=1, keepdims=True) + bdv,
            jnp.sum(ua * g, axis=1, keepdims=True) + bdv], axis=1)
        reta_ref[...] = jnp.concatenate([
            jnp.sum(ua * ga, axis=1, keepdims=True) + bdv,
            jnp.sum(u * ga, axis=1, keepdims=True) + bdv], axis=1)

    def body(a1_ref, a2_ref, y1a_ref, y1b_ref, y2a_ref, y2b_ref,
             w2a_ref, b2a_ref, wd1_ref, bd1_ref,
             w2b_ref, b2b_ref, wd2_ref, bd2_ref,
             m1_ref, m2_ref, aw_ref, au_ref,
             dwa_ref, dba_ref, dwb_ref, dbb_ref, b1a_ref, b1b_ref,
             e1_o, e2_o, ret1_o, reta1_o, ret2_o, reta2_o,
             alpha_o, comb_o, g1_o, g2_o,
             p1_s, p2_s,
             z1a_s, z1b_s, z2a_s, z2b_s,
             e1a_s, e1b_s, e2a_s, e2b_s,
             dvr1_s, dvc1_s, dvr2_s, dvc2_s):
        ph = pl.program_id(0)
        m = pl.program_id(1)
        rows = pl.ds(m * bm, bm)
        c1_s, c2_s, h1_s, h2_s = z1a_s, z1b_s, z2a_s, z2b_s

        @pl.when(ph == 0)
        def _prep():
            cols = pl.ds(m * bm, bm)
            for a_ref, p_s, dvr_s, dvc_s in (
                    (a1_ref, p1_s, dvr1_s, dvc1_s),
                    (a2_ref, p2_s, dvr2_s, dvc2_s)):
                t = a_ref[...]
                b = jnp.where(t != 0, 1.0, 0.0)
                rid = lax.broadcasted_iota(jnp.int32, t.shape, 0)
                cid = m * bm + lax.broadcasted_iota(jnp.int32, t.shape, 1)
                d = jnp.where(rid == cid, 1.0, b)
                p_s[m] = d.astype(BF16)
                dvr_s[0:1, cols] = 1.0 / jnp.sqrt(
                    jnp.sum(d, axis=0, keepdims=True))
                deg = lax.dot_general(d, jnp.ones((N, 1), F32), _TN,
                                      preferred_element_type=F32)
                dvc_s[cols, :] = 1.0 / jnp.sqrt(deg)

        @pl.when(ph == 1)
        def _norm():
            cols = pl.ds(m * bm, bm)
            for p_s, dvr_s, dvc_s in (
                    (p1_s, dvr1_s, dvc1_s),
                    (p2_s, dvr2_s, dvc2_s)):
                dr = dvr_s[0:1, cols]
                dc = dvc_s[...]
                p_s[m] = ((dc * p_s[m].astype(F32)) * dr).astype(BF16)

        @pl.when(ph == 2)
        def _l1():
            for p_s, ys, b_ref, zs in (
                    (p1_s, (y1a_ref, y1b_ref), b1a_ref, (z1a_s, z1b_s)),
                    (p2_s, (y2a_ref, y2b_ref), b1b_ref, (z2a_s, z2b_s))):
                s = p_s[m]
                for y_ref, z_ref in zip(ys, zs):
                    r = lax.dot_general(s, y_ref[...], _TN,
                                        preferred_element_type=F32)
                    z_ref[rows, :] = jax.nn.relu(r + b_ref[...]).astype(BF16)

        @pl.when(ph == 3)
        def _l2():
            for p_s, zs, w_ref, b_ref, es, e_out in (
                    (p1_s, (z1a_s, z1b_s), w2a_ref, b2a_ref,
                     (e1a_s, e1b_s), e1_o),
                    (p2_s, (z2a_s, z2b_s), w2b_ref, b2b_ref,
                     (e2a_s, e2b_s), e2_o)):
                s = p_s[m]
                w = w_ref[...].astype(BF16)
                for i, (z_ref, e_ref) in enumerate(zip(zs, es)):
                    q = lax.dot_general(z_ref[...], w, _NN,
                                        preferred_element_type=F32)
                    r = lax.dot_general(s, q.astype(BF16), _TN,
                                        preferred_element_type=F32)
                    r = r + b_ref[...]
                    if i == 0:
                        e_ref[rows, :] = r
                        e_out[...] = r
                    else:
                        e_ref[rows, :] = r.astype(BF16)

        @pl.when(ph == 4)
        def _readout():
            readout_one(m1_ref[...], e1a_s[...], e1b_s[...],
                        e1a_s[rows, :], e1b_s[rows, :], wd1_ref[...],
                        bd1_ref[0, 0], ret1_o, reta1_o)
            readout_one(m2_ref[...], e2a_s[...], e2b_s[...],
                        e2a_s[rows, :], e2b_s[rows, :], wd2_ref[...],
                        bd2_ref[0, 0], ret2_o, reta2_o)
            x1 = e1a_s[rows, :]
            x2 = e2a_s[rows, :]
            w = aw_ref[...]
            u = au_ref[...]
            v1 = jnp.tanh(jnp.dot(x1, w, preferred_element_type=F32))
            v2 = jnp.tanh(jnp.dot(x2, w, preferred_element_type=F32))
            s1 = jnp.sum(v1 * u, axis=1, keepdims=True) + 1e-6
            s2 = jnp.sum(v2 * u, axis=1, keepdims=True) + 1e-6
            mx = jnp.maximum(s1, s2)
            q1 = jnp.exp(s1 - mx)
            q2 = jnp.exp(s2 - mx)
            den = q1 + q2
            a1 = q1 / den
            a2 = q2 / den
            alpha_o[...] = jnp.concatenate([a1, a2], axis=1)
            comb = a1 * x1 + a2 * x2
            comb_o[...] = comb
            cb = comb.astype(BF16)
            c1_s[rows, :] = lax.dot_general(
                cb, dwa_ref[...].astype(BF16), _NN,
                preferred_element_type=F32).astype(BF16)
            c2_s[rows, :] = lax.dot_general(
                cb, dwb_ref[...].astype(BF16), _NN,
                preferred_element_type=F32).astype(BF16)

        @pl.when(ph == 5)
        def _h():
            for p_s, c_s, db_ref, h_s in (
                    (p1_s, c1_s, dba_ref, h1_s),
                    (p2_s, c2_s, dbb_ref, h2_s)):
                r = lax.dot_general(p_s[m], c_s[...], _TN,
                                    preferred_element_type=F32)
                h_s[rows, :] = jax.nn.relu(r + db_ref[...]).astype(BF16)

        @pl.when(ph == 6)
        def _g():
            for p_s, h_s, g_o in (
                    (p1_s, h1_s, g1_o),
                    (p2_s, h2_s, g2_o)):
                g_o[...] = lax.dot_general(
                    p_s[m], h_s[...], _TN,
                    preferred_element_type=F32).astype(BF16)

    small = [
        (w2a, (HID, OUT)), (b2a.reshape(1, OUT), (1, OUT)),
        (wd1, (OUT, OUT)), (bd1.reshape(1, 1), (1, 1)),
        (w2b, (HID, OUT)), (b2b.reshape(1, OUT), (1, OUT)),
        (wd2, (OUT, OUT)), (bd2.reshape(1, 1), (1, 1)),
    ]
    small2 = [
        (aw, (OUT, OUT)), (au, (1, OUT)),
        (dwa, (OUT, HID)), (dba.reshape(1, HID), (1, HID)),
        (dwb, (OUT, HID)), (dbb.reshape(1, HID), (1, HID)),
        (b1a.reshape(1, HID), (1, HID)), (b1b.reshape(1, HID), (1, HID)),
    ]
    in_specs = (
        [pl.BlockSpec((N, bm), p_idx)] * 2
        + [pl.BlockSpec((N, HID), const)] * 4
        + [pl.BlockSpec(shp, const) for _, shp in small]
        + [pl.BlockSpec((bm, N), gate(4))] * 2
        + [pl.BlockSpec(shp, const) for _, shp in small2]
    )
    args = ([adj1, adj2, y1a, y1b, y2a, y2b]
            + [a for a, _ in small] + [mask1, mask2]
            + [a for a, _ in small2])
    out_specs = [
        pl.BlockSpec((bm, OUT), gate(3)),
        pl.BlockSpec((bm, OUT), gate(3)),
        pl.BlockSpec((bm, 2), gate(4)),
        pl.BlockSpec((bm, 2), gate(4)),
        pl.BlockSpec((bm, 2), gate(4)),
        pl.BlockSpec((bm, 2), gate(4)),
        pl.BlockSpec((bm, 2), gate(4)),
        pl.BlockSpec((bm, OUT), gate(4)),
        pl.BlockSpec((bm, HID), gate(6)),
        pl.BlockSpec((bm, HID), gate(6)),
    ]
    out_shape = [
        jax.ShapeDtypeStruct((N, OUT), F32),
        jax.ShapeDtypeStruct((N, OUT), F32),
        jax.ShapeDtypeStruct((N, 2), F32),
        jax.ShapeDtypeStruct((N, 2), F32),
        jax.ShapeDtypeStruct((N, 2), F32),
        jax.ShapeDtypeStruct((N, 2), F32),
        jax.ShapeDtypeStruct((N, 2), F32),
        jax.ShapeDtypeStruct((N, OUT), F32),
        jax.ShapeDtypeStruct((N, HID), BF16),
        jax.ShapeDtypeStruct((N, HID), BF16),
    ]
    scratch_shapes = (
        [pltpu.VMEM((nb, N, bm), BF16)] * 2
        + [pltpu.VMEM((N, HID), BF16)] * 4
        + [pltpu.VMEM((N, OUT), F32),
           pltpu.VMEM((N, OUT), BF16),
           pltpu.VMEM((N, OUT), F32),
           pltpu.VMEM((N, OUT), BF16)]
        + [pltpu.VMEM((1, N), F32), pltpu.VMEM((N, 1), F32)] * 2
    )
    return pl.pallas_call(
        body,
        grid=(7, nb),
        in_specs=in_specs,
        out_specs=out_specs,
        out_shape=out_shape,
        scratch_shapes=scratch_shapes,
    )(*args)


def _ptmm2(ps, yss, biases=None, act=None, out_dtype=F32, bm=512,
           pre_ws=None):
    ng = len(ps)
    counts = [len(ys) for ys in yss]
    kcs = [ys[0].shape[1] for ys in yss]
    if pre_ws is not None:
        ncs = [w.shape[1] for w in pre_ws]
    else:
        ncs = kcs

    def body(*refs):
        refs = list(refs)
        p_refs = [refs.pop(0) for _ in range(ng)]
        y_refs = [[refs.pop(0) for _ in range(counts[g])] for g in range(ng)]
        w_refs = ([refs.pop(0) for _ in range(ng)]
                  if pre_ws is not None else [None] * ng)
        b_refs = ([refs.pop(0) for _ in range(ng)]
                  if biases is not None else [None] * ng)
        for g in range(ng):
            s = p_refs[g][...]
            if pre_ws is not None:
                w = w_refs[g][...].astype(BF16)
            for i in range(counts[g]):
                y = y_refs[g][i][...]
                if pre_ws is not None:
                    y = lax.dot_general(y, w, _NN,
                                        preferred_element_type=F32)
                    y = y.astype(BF16)
                r = lax.dot_general(s, y, _TN,
                                    preferred_element_type=F32)
                if biases is not None:
                    r = r + b_refs[g][...]
                if act is not None:
                    r = act(r)
                refs.pop(0)[...] = r.astype(out_dtype)

    in_specs = [pl.BlockSpec((N, bm), lambda m: (0, m))] * ng
    args = list(ps)
    for g in range(ng):
        in_specs += [pl.BlockSpec((N, kcs[g]), lambda m: (0, 0))] * counts[g]
        args += list(yss[g])
    if pre_ws is not None:
        for g in range(ng):
            in_specs.append(
                pl.BlockSpec((kcs[g], ncs[g]), lambda m: (0, 0)))
            args.append(pre_ws[g])
    if biases is not None:
        for g in range(ng):
            in_specs.append(pl.BlockSpec((1, ncs[g]), lambda m: (0, 0)))
            args.append(biases[g].reshape(1, ncs[g]))
    out_specs = []
    out_shape = []
    for g in range(ng):
        out_specs += [pl.BlockSpec((bm, ncs[g]), lambda m: (m, 0))] * counts[g]
        out_shape += [jax.ShapeDtypeStruct((N, ncs[g]), out_dtype)] * counts[g]
    flat = pl.pallas_call(
        body,
        grid=(N // bm,),
        in_specs=in_specs,
        out_specs=out_specs,
        out_shape=out_shape,
    )(*args)
    outs = []
    k = 0
    for g in range(ng):
        outs.append(list(flat[k:k + counts[g]]))
        k += counts[g]
    return outs


def _mm(avs, bmat, bias=None, act=None, out_dtype=F32, bm=512, bn=512):
    nd = len(avs)
    m_, k_ = avs[0].shape
    nc = bmat.shape[1]
    bn = min(bn, -(-nc // 128) * 128)
    grid = (m_ // bm, -(-nc // bn))
    k_main = (k_ // 128) * 128
    ragged = k_main != k_
    ktail_blk = k_main // 128

    def body(*refs):
        refs = list(refs)
        a_refs = [refs.pop(0) for _ in range(nd)]
        if ragged:
            at_refs = [refs.pop(0) for _ in range(nd)]
        b_ref = refs.pop(0)
        if ragged:
            bt_ref = refs.pop(0)
        if bias is not None:
            bias_ref = refs.pop(0)
        o_refs = refs
        bmain = b_ref[...].astype(BF16)
        if ragged:
            kid = lax.broadcasted_iota(jnp.int32, (128, bn), 0)
            btail = jnp.where(kid < (k_ - k_main), bt_ref[...], 0.0)
            btail = btail.astype(BF16)
        for i in range(nd):
            r = lax.dot_general(a_refs[i][...].astype(BF16), bmain, _NN,
                                preferred_element_type=F32)
            if ragged:
                kida = lax.broadcasted_iota(jnp.int32, (bm, 128), 1)
                atail = jnp.where(kida < (k_ - k_main), at_refs[i][...], 0.0)
                r = r + lax.dot_general(atail.astype(BF16), btail, _NN,
                                        preferred_element_type=F32)
            if bias is not None:
                r = r + bias_ref[...]
            if act is not None:
                r = act(r)
            o_refs[i][...] = r.astype(out_dtype)

    in_specs = [pl.BlockSpec((bm, k_main), lambda m, n: (m, 0))] * nd
    args = list(avs)
    if ragged:
        in_specs += [pl.BlockSpec((bm, 128),
                                  lambda m, n: (m, ktail_blk))] * nd
        args += list(avs)
    in_specs.append(pl.BlockSpec((k_main, bn), lambda m, n: (0, n)))
    args.append(bmat)
    if ragged:
        in_specs.append(pl.BlockSpec((128, bn), lambda m, n: (ktail_blk, n)))
        args.append(bmat)
    if bias is not None:
        in_specs.append(pl.BlockSpec((1, bn), lambda m, n: (0, n)))
        args.append(bias)
    return pl.pallas_call(
        body,
        grid=grid,
        in_specs=in_specs,
        out_specs=[pl.BlockSpec((bm, bn), lambda m, n: (m, n))] * nd,
        out_shape=[jax.ShapeDtypeStruct((m_, nc), out_dtype)] * nd,
        compiler_params=_PARAMS,
    )(*args)


def _readout_disc_attn(groups, aw, u_row, dec_w1s):
    bmr = 512
    nh = dec_w1s[0].shape[1]

    def one(mk, ef, eaf, er, ear, w, bdv, ret_ref, reta_ref):
        rs = jnp.sum(mk, axis=1, keepdims=True)
        mb = mk.astype(BF16)
        vs1 = lax.dot_general(mb, ef.astype(BF16), _NN,
                              preferred_element_type=F32)
        vs2 = lax.dot_general(mb, eaf.astype(BF16), _NN,
                              preferred_element_type=F32)
        gp = vs1 / rs
        gap = vs2 / rs
        n1 = jnp.maximum(jnp.sqrt(jnp.sum(gp * gp, axis=1, keepdims=True)),
                         1e-12)
        n2 = jnp.maximum(jnp.sqrt(jnp.sum(gap * gap, axis=1, keepdims=True)),
                         1e-12)
        g = jax.nn.sigmoid(gp / n1)
        ga = jax.nn.sigmoid(gap / n2)
        u = jnp.dot(er, w, preferred_element_type=F32)
        ua = jnp.dot(ear, w, preferred_element_type=F32)
        ret_ref[...] = jnp.concatenate([
            jnp.sum(u * g, axis=1, keepdims=True) + bdv,
            jnp.sum(ua * g, axis=1, keepdims=True) + bdv], axis=1)
        reta_ref[...] = jnp.concatenate([
            jnp.sum(ua * ga, axis=1, keepdims=True) + bdv,
            jnp.sum(u * ga, axis=1, keepdims=True) + bdv], axis=1)

    def body(m1, e1f, e1af, e1r, e1ar, w1, b1,
             m2, e2f, e2af, e2r, e2ar, w2, b2,
             aw_ref, u_ref, dw1_ref, dw2_ref,
             ret1, reta1, ret2, reta2, alpha_ref, comb_ref, c1_ref, c2_ref):
        one(m1[...], e1f[...], e1af[...], e1r[...], e1ar[...], w1[...],
            b1[0, 0], ret1, reta1)
        one(m2[...], e2f[...], e2af[...], e2r[...], e2ar[...], w2[...],
            b2[0, 0], ret2, reta2)
        x1 = e1r[...]
        x2 = e2r[...]
        w = aw_ref[...]
        u = u_ref[...]
        v1 = jnp.tanh(jnp.dot(x1, w, preferred_element_type=F32))
        v2 = jnp.tanh(jnp.dot(x2, w, preferred_element_type=F32))
        s1 = jnp.sum(v1 * u, axis=1, keepdims=True) + 1e-6
        s2 = jnp.sum(v2 * u, axis=1, keepdims=True) + 1e-6
        mx = jnp.maximum(s1, s2)
        p1 = jnp.exp(s1 - mx)
        p2 = jnp.exp(s2 - mx)
        den = p1 + p2
        a1 = p1 / den
        a2 = p2 / den
        alpha_ref[...] = jnp.concatenate([a1, a2], axis=1)
        comb = a1 * x1 + a2 * x2
        comb_ref[...] = comb
        cb = comb.astype(BF16)
        c1_ref[...] = lax.dot_general(
            cb, dw1_ref[...].astype(BF16), _NN,
            preferred_element_type=F32).astype(BF16)
        c2_ref[...] = lax.dot_general(
            cb, dw2_ref[...].astype(BF16), _NN,
            preferred_element_type=F32).astype(BF16)

    specs_one = [
        pl.BlockSpec((bmr, N), lambda m: (m, 0)),
        pl.BlockSpec((N, OUT), lambda m: (0, 0)),
        pl.BlockSpec((N, OUT), lambda m: (0, 0)),
        pl.BlockSpec((bmr, OUT), lambda m: (m, 0)),
        pl.BlockSpec((bmr, OUT), lambda m: (m, 0)),
        pl.BlockSpec((OUT, OUT), lambda m: (0, 0)),
        pl.BlockSpec((1, 1), lambda m: (0, 0)),
    ]
    args = []
    for mask, ec, es, wd, bd in groups:
        args += [mask, ec, es, ec, es, wd, bd.reshape(1, 1)]
    args += [aw, u_row, dec_w1s[0], dec_w1s[1]]
    return pl.pallas_call(
        body,
        grid=(N // bmr,),
        in_specs=specs_one + specs_one + [
            pl.BlockSpec((OUT, OUT), lambda m: (0, 0)),
            pl.BlockSpec((1, OUT), lambda m: (0, 0)),
            pl.BlockSpec((OUT, nh), lambda m: (0, 0)),
            pl.BlockSpec((OUT, nh), lambda m: (0, 0)),
        ],
        out_specs=[pl.BlockSpec((bmr, 2), lambda m: (m, 0))] * 4 + [
            pl.BlockSpec((bmr, 2), lambda m: (m, 0)),
            pl.BlockSpec((bmr, OUT), lambda m: (m, 0)),
            pl.BlockSpec((bmr, nh), lambda m: (m, 0)),
            pl.BlockSpec((bmr, nh), lambda m: (m, 0)),
        ],
        out_shape=[jax.ShapeDtypeStruct((N, 2), F32)] * 4 + [
            jax.ShapeDtypeStruct((N, 2), F32),
            jax.ShapeDtypeStruct((N, OUT), F32),
            jax.ShapeDtypeStruct((N, nh), BF16),
            jax.ShapeDtypeStruct((N, nh), BF16),
        ],
    )(*args)


def kernel(omics1_feat_shuffle, omics2_feat_shuffle, omics1_feat, omics2_feat,
           omics1_adj, omics2_adj, omics1_graph_neigh, omics2_graph_neigh,
           o1_enc_W1, o1_enc_b1, o1_enc_W2, o1_enc_b2, o1_disc_W, o1_disc_b,
           o2_enc_W1, o2_enc_b1, o2_enc_W2, o2_enc_b2, o2_disc_W, o2_disc_b,
           o1_dec_W1, o1_dec_b1, o1_dec_W2, o1_dec_b2,
           o2_dec_W1, o2_dec_b1, o2_dec_W2, o2_dec_b2, att_w, att_u):
    y1a, y1b = _mm([omics1_feat, omics1_feat_shuffle], o1_enc_W1,
                   out_dtype=BF16)
    y2a, y2b = _mm([omics2_feat, omics2_feat_shuffle], o2_enc_W1,
                   out_dtype=BF16)

    (e1a, e2a, o1_ret, o1_ret_a, o2_ret, o2_ret_a,
     alpha, comb, g1, g2) = _core(
        omics1_adj, omics2_adj, y1a, y1b, y2a, y2b,
        o1_enc_W2, o1_enc_b2, o1_disc_W, o1_disc_b,
        o2_enc_W2, o2_enc_b2, o2_disc_W, o2_disc_b,
        omics1_graph_neigh, omics2_graph_neigh,
        att_w, att_u.reshape(1, OUT),
        o1_dec_W1, o1_dec_b1, o2_dec_W1, o2_dec_b1,
        o1_enc_b1, o2_enc_b1)

    (o1_rec,) = _mm([g1], o1_dec_W2, bias=o1_dec_b2.reshape(1, -1))
    (o2_rec,) = _mm([g2], o2_dec_W2, bias=o2_dec_b2.reshape(1, -1))

    return (e1a, o1_rec, o1_ret, o1_ret_a,
            e2a, o2_rec, o2_ret, o2_ret_a, comb, alpha)


# speedup vs baseline: 1.3395x; 1.0609x over previous
import jax
import jax.numpy as jnp
from jax import lax
from jax.experimental import pallas as pl
from jax.experimental.pallas import tpu as pltpu

N = 2048
HID = 256
OUT = 128
F32 = jnp.float32
BF16 = jnp.bfloat16

_TN = (((0,), (0,)), ((), ()))
_NN = (((1,), (0,)), ((), ()))
_PARAMS = pltpu.CompilerParams(
    dimension_semantics=("parallel", "parallel"))


def _core(adj1, adj2, y1a, y1b, y2a, y2b,
          w2a, b2a, wd1, bd1, w2b, b2b, wd2, bd2,
          mask1, mask2, aw, au, dwa, dba, dwb, dbb,
          b1a, b1b):
    bm = 256
    nb = N // bm

    def gate(phx):
        return lambda ph, m: (jnp.where(ph < phx, 0,
                                        jnp.where(ph == phx, m, nb - 1)), 0)

    def const(ph, m):
        return (0, 0)

    p_idx = lambda ph, m: (0, jnp.where(ph == 0, m, nb - 1))

    def readout_one(mk, ef, eaf_bf, er, ear_bf, w, bdv, ret_ref, reta_ref):
        rs = jnp.sum(mk, axis=1, keepdims=True)
        mb = mk.astype(BF16)
        vs1 = lax.dot_general(mb, ef.astype(BF16), _NN,
                              preferred_element_type=F32)
        vs2 = lax.dot_general(mb, eaf_bf, _NN,
                              preferred_element_type=F32)
        ear = ear_bf.astype(F32)
        gp = vs1 / rs
        gap = vs2 / rs
        n1 = jnp.maximum(jnp.sqrt(jnp.sum(gp * gp, axis=1, keepdims=True)),
                         1e-12)
        n2 = jnp.maximum(jnp.sqrt(jnp.sum(gap * gap, axis=1, keepdims=True)),
                         1e-12)
        g = jax.nn.sigmoid(gp / n1)
        ga = jax.nn.sigmoid(gap / n2)
        u = jnp.dot(er, w, preferred_element_type=F32)
        ua = jnp.dot(ear, w, preferred_element_type=F32)
        ret_ref[...] = jnp.concatenate([
            jnp.sum(u * g, axis=1, keepdims=True) + bdv,
            jnp.sum(ua * g, axis=1, keepdims=True) + bdv], axis=1)
        reta_ref[...] = jnp.concatenate([
            jnp.sum(ua * ga, axis=1, keepdims=True) + bdv,
            jnp.sum(u * ga, axis=1, keepdims=True) + bdv], axis=1)

    def body(a1_ref, a2_ref, y1a_ref, y1b_ref, y2a_ref, y2b_ref,
             w2a_ref, b2a_ref, wd1_ref, bd1_ref,
             w2b_ref, b2b_ref, wd2_ref, bd2_ref,
             m1_ref, m2_ref, aw_ref, au_ref,
             dwa_ref, dba_ref, dwb_ref, dbb_ref, b1a_ref, b1b_ref,
             e1_o, e2_o, ret1_o, reta1_o, ret2_o, reta2_o,
             alpha_o, comb_o, g1_o, g2_o,
             p1_s, p2_s,
             z1a_s, z1b_s, z2a_s, z2b_s,
             e1a_s, e1b_s, e2a_s, e2b_s,
             dvr1_s, dvc1_s, dvr2_s, dvc2_s):
        ph = pl.program_id(0)
        m = pl.program_id(1)
        rows = pl.ds(m * bm, bm)
        c1_s, c2_s, h1_s, h2_s = z1a_s, z1b_s, z2a_s, z2b_s

        @pl.when(ph == 0)
        def _prep():
            cols = pl.ds(m * bm, bm)
            for a_ref, p_s, dvr_s, dvc_s in (
                    (a1_ref, p1_s, dvr1_s, dvc1_s),
                    (a2_ref, p2_s, dvr2_s, dvc2_s)):
                t = a_ref[...]
                b = jnp.where(t != 0, 1.0, 0.0)
                rid = lax.broadcasted_iota(jnp.int32, t.shape, 0)
                cid = m * bm + lax.broadcasted_iota(jnp.int32, t.shape, 1)
                d = jnp.where(rid == cid, 1.0, b)
                p_s[m] = d.astype(BF16)
                dvr_s[0:1, cols] = 1.0 / jnp.sqrt(
                    jnp.sum(d, axis=0, keepdims=True))
                deg = lax.dot_general(d, jnp.ones((N, 1), F32), _TN,
                                      preferred_element_type=F32)
                dvc_s[cols, :] = 1.0 / jnp.sqrt(deg)

        @pl.when(ph == 1)
        def _norm():
            cols = pl.ds(m * bm, bm)
            for p_s, dvr_s, dvc_s in (
                    (p1_s, dvr1_s, dvc1_s),
                    (p2_s, dvr2_s, dvc2_s)):
                dr = dvr_s[0:1, cols]
                dc = dvc_s[...]
                p_s[m] = ((dc * p_s[m].astype(F32)) * dr).astype(BF16)

        @pl.when(ph == 2)
        def _l1():
            for p_s, ys, b_ref, zs in (
                    (p1_s, (y1a_ref, y1b_ref), b1a_ref, (z1a_s, z1b_s)),
                    (p2_s, (y2a_ref, y2b_ref), b1b_ref, (z2a_s, z2b_s))):
                s = p_s[m]
                for y_ref, z_ref in zip(ys, zs):
                    r = lax.dot_general(s, y_ref[...], _TN,
                                        preferred_element_type=F32)
                    z_ref[rows, :] = jax.nn.relu(r + b_ref[...]).astype(BF16)

        @pl.when(ph == 3)
        def _l2():
            for p_s, zs, w_ref, b_ref, es, e_out in (
                    (p1_s, (z1a_s, z1b_s), w2a_ref, b2a_ref,
                     (e1a_s, e1b_s), e1_o),
                    (p2_s, (z2a_s, z2b_s), w2b_ref, b2b_ref,
                     (e2a_s, e2b_s), e2_o)):
                s = p_s[m]
                w = w_ref[...].astype(BF16)
                for i, (z_ref, e_ref) in enumerate(zip(zs, es)):
                    q = lax.dot_general(z_ref[...], w, _NN,
                                        preferred_element_type=F32)
                    r = lax.dot_general(s, q.astype(BF16), _TN,
                                        preferred_element_type=F32)
                    r = r + b_ref[...]
                    if i == 0:
                        e_ref[rows, :] = r
                        e_out[...] = r
                    else:
                        e_ref[rows, :] = r.astype(BF16)

        @pl.when(ph == 4)
        def _readout():
            readout_one(m1_ref[...], e1a_s[...], e1b_s[...],
                        e1a_s[rows, :], e1b_s[rows, :], wd1_ref[...],
                        bd1_ref[0, 0], ret1_o, reta1_o)
            readout_one(m2_ref[...], e2a_s[...], e2b_s[...],
                        e2a_s[rows, :], e2b_s[rows, :], wd2_ref[...],
                        bd2_ref[0, 0], ret2_o, reta2_o)
            x1 = e1a_s[rows, :]
            x2 = e2a_s[rows, :]
            w = aw_ref[...]
            u = au_ref[...]
            v1 = jnp.tanh(jnp.dot(x1, w, preferred_element_type=F32))
            v2 = jnp.tanh(jnp.dot(x2, w, preferred_element_type=F32))
            s1 = jnp.sum(v1 * u, axis=1, keepdims=True) + 1e-6
            s2 = jnp.sum(v2 * u, axis=1, keepdims=True) + 1e-6
            mx = jnp.maximum(s1, s2)
            q1 = jnp.exp(s1 - mx)
            q2 = jnp.exp(s2 - mx)
            den = q1 + q2
            a1 = q1 / den
            a2 = q2 / den
            alpha_o[...] = jnp.concatenate([a1, a2], axis=1)
            comb = a1 * x1 + a2 * x2
            comb_o[...] = comb
            cb = comb.astype(BF16)
            c1_s[rows, :] = lax.dot_general(
                cb, dwa_ref[...].astype(BF16), _NN,
                preferred_element_type=F32).astype(BF16)
            c2_s[rows, :] = lax.dot_general(
                cb, dwb_ref[...].astype(BF16), _NN,
                preferred_element_type=F32).astype(BF16)

        @pl.when(ph == 5)
        def _h():
            for p_s, c_s, db_ref, h_s in (
                    (p1_s, c1_s, dba_ref, h1_s),
                    (p2_s, c2_s, dbb_ref, h2_s)):
                r = lax.dot_general(p_s[m], c_s[...], _TN,
                                    preferred_element_type=F32)
                h_s[rows, :] = jax.nn.relu(r + db_ref[...]).astype(BF16)

        @pl.when(ph == 6)
        def _g():
            for p_s, h_s, g_o in (
                    (p1_s, h1_s, g1_o),
                    (p2_s, h2_s, g2_o)):
                g_o[...] = lax.dot_general(
                    p_s[m], h_s[...], _TN,
                    preferred_element_type=F32).astype(BF16)

    small = [
        (w2a, (HID, OUT)), (b2a.reshape(1, OUT), (1, OUT)),
        (wd1, (OUT, OUT)), (bd1.reshape(1, 1), (1, 1)),
        (w2b, (HID, OUT)), (b2b.reshape(1, OUT), (1, OUT)),
        (wd2, (OUT, OUT)), (bd2.reshape(1, 1), (1, 1)),
    ]
    small2 = [
        (aw, (OUT, OUT)), (au, (1, OUT)),
        (dwa, (OUT, HID)), (dba.reshape(1, HID), (1, HID)),
        (dwb, (OUT, HID)), (dbb.reshape(1, HID), (1, HID)),
        (b1a.reshape(1, HID), (1, HID)), (b1b.reshape(1, HID), (1, HID)),
    ]
    in_specs = (
        [pl.BlockSpec((N, bm), p_idx)] * 2
        + [pl.BlockSpec((N, HID), const)] * 4
        + [pl.BlockSpec(shp, const) for _, shp in small]
        + [pl.BlockSpec((bm, N), gate(4))] * 2
        + [pl.BlockSpec(shp, const) for _, shp in small2]
    )
    args = ([adj1, adj2, y1a, y1b, y2a, y2b]
            + [a for a, _ in small] + [mask1, mask2]
            + [a for a, _ in small2])
    out_specs = [
        pl.BlockSpec((bm, OUT), gate(3)),
        pl.BlockSpec((bm, OUT), gate(3)),
        pl.BlockSpec((bm, 2), gate(4)),
        pl.BlockSpec((bm, 2), gate(4)),
        pl.BlockSpec((bm, 2), gate(4)),
        pl.BlockSpec((bm, 2), gate(4)),
        pl.BlockSpec((bm, 2), gate(4)),
        pl.BlockSpec((bm, OUT), gate(4)),
        pl.BlockSpec((bm, HID), gate(6)),
        pl.BlockSpec((bm, HID), gate(6)),
    ]
    out_shape = [
        jax.ShapeDtypeStruct((N, OUT), F32),
        jax.ShapeDtypeStruct((N, OUT), F32),
        jax.ShapeDtypeStruct((N, 2), F32),
        jax.ShapeDtypeStruct((N, 2), F32),
        jax.ShapeDtypeStruct((N, 2), F32),
        jax.ShapeDtypeStruct((N, 2), F32),
        jax.ShapeDtypeStruct((N, 2), F32),
        jax.ShapeDtypeStruct((N, OUT), F32),
        jax.ShapeDtypeStruct((N, HID), BF16),
        jax.ShapeDtypeStruct((N, HID), BF16),
    ]
    scratch_shapes = (
        [pltpu.VMEM((nb, N, bm), BF16)] * 2
        + [pltpu.VMEM((N, HID), BF16)] * 4
        + [pltpu.VMEM((N, OUT), F32),
           pltpu.VMEM((N, OUT), BF16),
           pltpu.VMEM((N, OUT), F32),
           pltpu.VMEM((N, OUT), BF16)]
        + [pltpu.VMEM((1, N), F32), pltpu.VMEM((N, 1), F32)] * 2
    )
    return pl.pallas_call(
        body,
        grid=(7, nb),
        in_specs=in_specs,
        out_specs=out_specs,
        out_shape=out_shape,
        scratch_shapes=scratch_shapes,
    )(*args)


def _ptmm2(ps, yss, biases=None, act=None, out_dtype=F32, bm=512,
           pre_ws=None):
    ng = len(ps)
    counts = [len(ys) for ys in yss]
    kcs = [ys[0].shape[1] for ys in yss]
    if pre_ws is not None:
        ncs = [w.shape[1] for w in pre_ws]
    else:
        ncs = kcs

    def body(*refs):
        refs = list(refs)
        p_refs = [refs.pop(0) for _ in range(ng)]
        y_refs = [[refs.pop(0) for _ in range(counts[g])] for g in range(ng)]
        w_refs = ([refs.pop(0) for _ in range(ng)]
                  if pre_ws is not None else [None] * ng)
        b_refs = ([refs.pop(0) for _ in range(ng)]
                  if biases is not None else [None] * ng)
        for g in range(ng):
            s = p_refs[g][...]
            if pre_ws is not None:
                w = w_refs[g][...].astype(BF16)
            for i in range(counts[g]):
                y = y_refs[g][i][...]
                if pre_ws is not None:
                    y = lax.dot_general(y, w, _NN,
                                        preferred_element_type=F32)
                    y = y.astype(BF16)
                r = lax.dot_general(s, y, _TN,
                                    preferred_element_type=F32)
                if biases is not None:
                    r = r + b_refs[g][...]
                if act is not None:
                    r = act(r)
                refs.pop(0)[...] = r.astype(out_dtype)

    in_specs = [pl.BlockSpec((N, bm), lambda m: (0, m))] * ng
    args = list(ps)
    for g in range(ng):
        in_specs += [pl.BlockSpec((N, kcs[g]), lambda m: (0, 0))] * counts[g]
        args += list(yss[g])
    if pre_ws is not None:
        for g in range(ng):
            in_specs.append(
                pl.BlockSpec((kcs[g], ncs[g]), lambda m: (0, 0)))
            args.append(pre_ws[g])
    if biases is not None:
        for g in range(ng):
            in_specs.append(pl.BlockSpec((1, ncs[g]), lambda m: (0, 0)))
            args.append(biases[g].reshape(1, ncs[g]))
    out_specs = []
    out_shape = []
    for g in range(ng):
        out_specs += [pl.BlockSpec((bm, ncs[g]), lambda m: (m, 0))] * counts[g]
        out_shape += [jax.ShapeDtypeStruct((N, ncs[g]), out_dtype)] * counts[g]
    flat = pl.pallas_call(
        body,
        grid=(N // bm,),
        in_specs=in_specs,
        out_specs=out_specs,
        out_shape=out_shape,
    )(*args)
    outs = []
    k = 0
    for g in range(ng):
        outs.append(list(flat[k:k + counts[g]]))
        k += counts[g]
    return outs


def _mm(avs, bmat, bias=None, act=None, out_dtype=F32, bm=512, bn=512):
    nd = len(avs)
    m_, k_ = avs[0].shape
    nc = bmat.shape[1]
    bn = min(bn, -(-nc // 128) * 128)
    grid = (m_ // bm, -(-nc // bn))
    k_main = (k_ // 128) * 128
    ragged = k_main != k_
    ktail_blk = k_main // 128

    def body(*refs):
        refs = list(refs)
        a_refs = [refs.pop(0) for _ in range(nd)]
        if ragged:
            at_refs = [refs.pop(0) for _ in range(nd)]
        b_ref = refs.pop(0)
        if ragged:
            bt_ref = refs.pop(0)
        if bias is not None:
            bias_ref = refs.pop(0)
        o_refs = refs
        bmain = b_ref[...].astype(BF16)
        if ragged:
            kid = lax.broadcasted_iota(jnp.int32, (128, bn), 0)
            btail = jnp.where(kid < (k_ - k_main), bt_ref[...], 0.0)
            btail = btail.astype(BF16)
        for i in range(nd):
            r = lax.dot_general(a_refs[i][...].astype(BF16), bmain, _NN,
                                preferred_element_type=F32)
            if ragged:
                kida = lax.broadcasted_iota(jnp.int32, (bm, 128), 1)
                atail = jnp.where(kida < (k_ - k_main), at_refs[i][...], 0.0)
                r = r + lax.dot_general(atail.astype(BF16), btail, _NN,
                                        preferred_element_type=F32)
            if bias is not None:
                r = r + bias_ref[...]
            if act is not None:
                r = act(r)
            o_refs[i][...] = r.astype(out_dtype)

    in_specs = [pl.BlockSpec((bm, k_main), lambda m, n: (m, 0))] * nd
    args = list(avs)
    if ragged:
        in_specs += [pl.BlockSpec((bm, 128),
                                  lambda m, n: (m, ktail_blk))] * nd
        args += list(avs)
    in_specs.append(pl.BlockSpec((k_main, bn), lambda m, n: (0, n)))
    args.append(bmat)
    if ragged:
        in_specs.append(pl.BlockSpec((128, bn), lambda m, n: (ktail_blk, n)))
        args.append(bmat)
    if bias is not None:
        in_specs.append(pl.BlockSpec((1, bn), lambda m, n: (0, n)))
        args.append(bias)
    return pl.pallas_call(
        body,
        grid=grid,
        in_specs=in_specs,
        out_specs=[pl.BlockSpec((bm, bn), lambda m, n: (m, n))] * nd,
        out_shape=[jax.ShapeDtypeStruct((m_, nc), out_dtype)] * nd,
        compiler_params=_PARAMS,
    )(*args)


def _readout_disc_attn(groups, aw, u_row, dec_w1s):
    bmr = 512
    nh = dec_w1s[0].shape[1]

    def one(mk, ef, eaf, er, ear, w, bdv, ret_ref, reta_ref):
        rs = jnp.sum(mk, axis=1, keepdims=True)
        mb = mk.astype(BF16)
        vs1 = lax.dot_general(mb, ef.astype(BF16), _NN,
                              preferred_element_type=F32)
        vs2 = lax.dot_general(mb, eaf.astype(BF16), _NN,
                              preferred_element_type=F32)
        gp = vs1 / rs
        gap = vs2 / rs
        n1 = jnp.maximum(jnp.sqrt(jnp.sum(gp * gp, axis=1, keepdims=True)),
                         1e-12)
        n2 = jnp.maximum(jnp.sqrt(jnp.sum(gap * gap, axis=1, keepdims=True)),
                         1e-12)
        g = jax.nn.sigmoid(gp / n1)
        ga = jax.nn.sigmoid(gap / n2)
        u = jnp.dot(er, w, preferred_element_type=F32)
        ua = jnp.dot(ear, w, preferred_element_type=F32)
        ret_ref[...] = jnp.concatenate([
            jnp.sum(u * g, axis=1, keepdims=True) + bdv,
            jnp.sum(ua * g, axis=1, keepdims=True) + bdv], axis=1)
        reta_ref[...] = jnp.concatenate([
            jnp.sum(ua * ga, axis=1, keepdims=True) + bdv,
            jnp.sum(u * ga, axis=1, keepdims=True) + bdv], axis=1)

    def body(m1, e1f, e1af, e1r, e1ar, w1, b1,
             m2, e2f, e2af, e2r, e2ar, w2, b2,
             aw_ref, u_ref, dw1_ref, dw2_ref,
             ret1, reta1, ret2, reta2, alpha_ref, comb_ref, c1_ref, c2_ref):
        one(m1[...], e1f[...], e1af[...], e1r[...], e1ar[...], w1[...],
            b1[0, 0], ret1, reta1)
        one(m2[...], e2f[...], e2af[...], e2r[...], e2ar[...], w2[...],
            b2[0, 0], ret2, reta2)
        x1 = e1r[...]
        x2 = e2r[...]
        w = aw_ref[...]
        u = u_ref[...]
        v1 = jnp.tanh(jnp.dot(x1, w, preferred_element_type=F32))
        v2 = jnp.tanh(jnp.dot(x2, w, preferred_element_type=F32))
        s1 = jnp.sum(v1 * u, axis=1, keepdims=True) + 1e-6
        s2 = jnp.sum(v2 * u, axis=1, keepdims=True) + 1e-6
        mx = jnp.maximum(s1, s2)
        p1 = jnp.exp(s1 - mx)
        p2 = jnp.exp(s2 - mx)
        den = p1 + p2
        a1 = p1 / den
        a2 = p2 / den
        alpha_ref[...] = jnp.concatenate([a1, a2], axis=1)
        comb = a1 * x1 + a2 * x2
        comb_ref[...] = comb
        cb = comb.astype(BF16)
        c1_ref[...] = lax.dot_general(
            cb, dw1_ref[...].astype(BF16), _NN,
            preferred_element_type=F32).astype(BF16)
        c2_ref[...] = lax.dot_general(
            cb, dw2_ref[...].astype(BF16), _NN,
            preferred_element_type=F32).astype(BF16)

    specs_one = [
        pl.BlockSpec((bmr, N), lambda m: (m, 0)),
        pl.BlockSpec((N, OUT), lambda m: (0, 0)),
        pl.BlockSpec((N, OUT), lambda m: (0, 0)),
        pl.BlockSpec((bmr, OUT), lambda m: (m, 0)),
        pl.BlockSpec((bmr, OUT), lambda m: (m, 0)),
        pl.BlockSpec((OUT, OUT), lambda m: (0, 0)),
        pl.BlockSpec((1, 1), lambda m: (0, 0)),
    ]
    args = []
    for mask, ec, es, wd, bd in groups:
        args += [mask, ec, es, ec, es, wd, bd.reshape(1, 1)]
    args += [aw, u_row, dec_w1s[0], dec_w1s[1]]
    return pl.pallas_call(
        body,
        grid=(N // bmr,),
        in_specs=specs_one + specs_one + [
            pl.BlockSpec((OUT, OUT), lambda m: (0, 0)),
            pl.BlockSpec((1, OUT), lambda m: (0, 0)),
            pl.BlockSpec((OUT, nh), lambda m: (0, 0)),
            pl.BlockSpec((OUT, nh), lambda m: (0, 0)),
        ],
        out_specs=[pl.BlockSpec((bmr, 2), lambda m: (m, 0))] * 4 + [
            pl.BlockSpec((bmr, 2), lambda m: (m, 0)),
            pl.BlockSpec((bmr, OUT), lambda m: (m, 0)),
            pl.BlockSpec((bmr, nh), lambda m: (m, 0)),
            pl.BlockSpec((bmr, nh), lambda m: (m, 0)),
        ],
        out_shape=[jax.ShapeDtypeStruct((N, 2), F32)] * 4 + [
            jax.ShapeDtypeStruct((N, 2), F32),
            jax.ShapeDtypeStruct((N, OUT), F32),
            jax.ShapeDtypeStruct((N, nh), BF16),
            jax.ShapeDtypeStruct((N, nh), BF16),
        ],
    )(*args)


def _mm_pair(a1, b1, bias1, a2, b2, bias2, bm=512):
    n1 = b1.shape[1]
    n2 = b2.shape[1]
    k = b1.shape[0]

    def body(a1_ref, b1_ref, s1_ref, a2_ref, b2_ref, s2_ref,
             o1_ref, o2_ref):
        for a_ref, b_ref, s_ref, o_ref in (
                (a1_ref, b1_ref, s1_ref, o1_ref),
                (a2_ref, b2_ref, s2_ref, o2_ref)):
            r = lax.dot_general(a_ref[...].astype(BF16),
                                b_ref[...].astype(BF16), _NN,
                                preferred_element_type=F32)
            o_ref[...] = r + s_ref[...]

    def one(nc):
        return [
            pl.BlockSpec((bm, k), lambda m: (m, 0)),
            pl.BlockSpec((k, nc), lambda m: (0, 0)),
            pl.BlockSpec((1, nc), lambda m: (0, 0)),
        ]

    return pl.pallas_call(
        body,
        grid=(N // bm,),
        in_specs=one(n1) + one(n2),
        out_specs=[pl.BlockSpec((bm, n1), lambda m: (m, 0)),
                   pl.BlockSpec((bm, n2), lambda m: (m, 0))],
        out_shape=[jax.ShapeDtypeStruct((N, n1), F32),
                   jax.ShapeDtypeStruct((N, n2), F32)],
    )(a1, b1, bias1.reshape(1, n1), a2, b2, bias2.reshape(1, n2))


def kernel(omics1_feat_shuffle, omics2_feat_shuffle, omics1_feat, omics2_feat,
           omics1_adj, omics2_adj, omics1_graph_neigh, omics2_graph_neigh,
           o1_enc_W1, o1_enc_b1, o1_enc_W2, o1_enc_b2, o1_disc_W, o1_disc_b,
           o2_enc_W1, o2_enc_b1, o2_enc_W2, o2_enc_b2, o2_disc_W, o2_disc_b,
           o1_dec_W1, o1_dec_b1, o1_dec_W2, o1_dec_b2,
           o2_dec_W1, o2_dec_b1, o2_dec_W2, o2_dec_b2, att_w, att_u):
    y1a, y1b = _mm([omics1_feat, omics1_feat_shuffle], o1_enc_W1,
                   out_dtype=BF16)
    y2a, y2b = _mm([omics2_feat, omics2_feat_shuffle], o2_enc_W1,
                   out_dtype=BF16)

    (e1a, e2a, o1_ret, o1_ret_a, o2_ret, o2_ret_a,
     alpha, comb, g1, g2) = _core(
        omics1_adj, omics2_adj, y1a, y1b, y2a, y2b,
        o1_enc_W2, o1_enc_b2, o1_disc_W, o1_disc_b,
        o2_enc_W2, o2_enc_b2, o2_disc_W, o2_disc_b,
        omics1_graph_neigh, omics2_graph_neigh,
        att_w, att_u.reshape(1, OUT),
        o1_dec_W1, o1_dec_b1, o2_dec_W1, o2_dec_b1,
        o1_enc_b1, o2_enc_b1)

    o1_rec, o2_rec = _mm_pair(g1, o1_dec_W2, o1_dec_b2,
                              g2, o2_dec_W2, o2_dec_b2)

    return (e1a, o1_rec, o1_ret, o1_ret_a,
            e2a, o2_rec, o2_ret, o2_ret_a, comb, alpha)


# speedup vs baseline: 1.3441x; 1.0034x over previous
import jax
import jax.numpy as jnp
from jax import lax
from jax.experimental import pallas as pl
from jax.experimental.pallas import tpu as pltpu

N = 2048
HID = 256
OUT = 128
F32 = jnp.float32
BF16 = jnp.bfloat16

_TN = (((0,), (0,)), ((), ()))
_NN = (((1,), (0,)), ((), ()))
_PARAMS = pltpu.CompilerParams(
    dimension_semantics=("parallel", "parallel"))


def _core(adj1, adj2, y1a, y1b, y2a, y2b,
          w2a, b2a, wd1, bd1, w2b, b2b, wd2, bd2,
          mask1, mask2, aw, au, dwa, dba, dwb, dbb,
          b1a, b1b):
    bm = 256
    nb = N // bm

    def gate(phx):
        return lambda ph, m: (jnp.where(ph < phx, 0,
                                        jnp.where(ph == phx, m, nb - 1)), 0)

    def const(ph, m):
        return (0, 0)

    p_idx = lambda ph, m: (0, jnp.where(ph == 0, m, nb - 1))

    def readout_one(mk, ef, eaf_bf, er, ear_bf, w, bdv, ret_ref, reta_ref):
        rs = jnp.sum(mk, axis=1, keepdims=True)
        mb = mk.astype(BF16)
        vs1 = lax.dot_general(mb, ef.astype(BF16), _NN,
                              preferred_element_type=F32)
        vs2 = lax.dot_general(mb, eaf_bf, _NN,
                              preferred_element_type=F32)
        ear = ear_bf.astype(F32)
        gp = vs1 / rs
        gap = vs2 / rs
        n1 = jnp.maximum(jnp.sqrt(jnp.sum(gp * gp, axis=1, keepdims=True)),
                         1e-12)
        n2 = jnp.maximum(jnp.sqrt(jnp.sum(gap * gap, axis=1, keepdims=True)),
                         1e-12)
        g = jax.nn.sigmoid(gp / n1)
        ga = jax.nn.sigmoid(gap / n2)
        u = jnp.dot(er, w, preferred_element_type=F32)
        ua = jnp.dot(ear, w, preferred_element_type=F32)
        ret_ref[...] = jnp.concatenate([
            jnp.sum(u * g, axis=1, keepdims=True) + bdv,
            jnp.sum(ua * g, axis=1, keepdims=True) + bdv], axis=1)
        reta_ref[...] = jnp.concatenate([
            jnp.sum(ua * ga, axis=1, keepdims=True) + bdv,
            jnp.sum(u * ga, axis=1, keepdims=True) + bdv], axis=1)

    def body(a1_ref, a2_ref, y1a_ref, y1b_ref, y2a_ref, y2b_ref,
             w2a_ref, b2a_ref, wd1_ref, bd1_ref,
             w2b_ref, b2b_ref, wd2_ref, bd2_ref,
             m1_ref, m2_ref, aw_ref, au_ref,
             dwa_ref, dba_ref, dwb_ref, dbb_ref, b1a_ref, b1b_ref,
             e1_o, e2_o, ret1_o, reta1_o, ret2_o, reta2_o,
             alpha_o, comb_o, g1_o, g2_o,
             p1_s, p2_s,
             z1a_s, z1b_s, z2a_s, z2b_s,
             e1a_s, e1b_s, e2a_s, e2b_s,
             dvr1_s, dvc1_s, dvr2_s, dvc2_s):
        ph = pl.program_id(0)
        m = pl.program_id(1)
        rows = pl.ds(m * bm, bm)
        c1_s, c2_s, h1_s, h2_s = z1a_s, z1b_s, z2a_s, z2b_s

        @pl.when(ph == 0)
        def _prep():
            cols = pl.ds(m * bm, bm)
            for a_ref, p_s, dvr_s, dvc_s in (
                    (a1_ref, p1_s, dvr1_s, dvc1_s),
                    (a2_ref, p2_s, dvr2_s, dvc2_s)):
                t = a_ref[...]
                b = jnp.where(t != 0, 1.0, 0.0)
                rid = lax.broadcasted_iota(jnp.int32, t.shape, 0)
                cid = m * bm + lax.broadcasted_iota(jnp.int32, t.shape, 1)
                d = jnp.where(rid == cid, 1.0, b)
                p_s[m] = d.astype(BF16)
                dvr_s[0:1, cols] = 1.0 / jnp.sqrt(
                    jnp.sum(d, axis=0, keepdims=True))
                deg = lax.dot_general(d, jnp.ones((N, 1), F32), _TN,
                                      preferred_element_type=F32)
                dvc_s[cols, :] = 1.0 / jnp.sqrt(deg)

        @pl.when(ph == 1)
        def _norm():
            cols = pl.ds(m * bm, bm)
            for p_s, dvr_s, dvc_s in (
                    (p1_s, dvr1_s, dvc1_s),
                    (p2_s, dvr2_s, dvc2_s)):
                dr = dvr_s[0:1, cols]
                dc = dvc_s[...]
                p_s[m] = ((dc * p_s[m].astype(F32)) * dr).astype(BF16)

        @pl.when(ph == 2)
        def _l1():
            for p_s, ys, b_ref, zs in (
                    (p1_s, (y1a_ref, y1b_ref), b1a_ref, (z1a_s, z1b_s)),
                    (p2_s, (y2a_ref, y2b_ref), b1b_ref, (z2a_s, z2b_s))):
                s = p_s[m]
                for y_ref, z_ref in zip(ys, zs):
                    r = lax.dot_general(s, y_ref[...], _TN,
                                        preferred_element_type=F32)
                    z_ref[rows, :] = jax.nn.relu(r + b_ref[...]).astype(BF16)

        @pl.when(ph == 3)
        def _l2():
            for p_s, zs, w_ref, b_ref, es, e_out in (
                    (p1_s, (z1a_s, z1b_s), w2a_ref, b2a_ref,
                     (e1a_s, e1b_s), e1_o),
                    (p2_s, (z2a_s, z2b_s), w2b_ref, b2b_ref,
                     (e2a_s, e2b_s), e2_o)):
                s = p_s[m]
                w = w_ref[...].astype(BF16)
                for i, (z_ref, e_ref) in enumerate(zip(zs, es)):
                    q = lax.dot_general(z_ref[...], w, _NN,
                                        preferred_element_type=F32)
                    r = lax.dot_general(s, q.astype(BF16), _TN,
                                        preferred_element_type=F32)
                    r = r + b_ref[...]
                    if i == 0:
                        e_ref[rows, :] = r
                        e_out[...] = r
                    else:
                        e_ref[rows, :] = r.astype(BF16)

        @pl.when(ph == 4)
        def _readout():
            readout_one(m1_ref[...], e1a_s[...], e1b_s[...],
                        e1a_s[rows, :], e1b_s[rows, :], wd1_ref[...],
                        bd1_ref[0, 0], ret1_o, reta1_o)
            readout_one(m2_ref[...], e2a_s[...], e2b_s[...],
                        e2a_s[rows, :], e2b_s[rows, :], wd2_ref[...],
                        bd2_ref[0, 0], ret2_o, reta2_o)
            x1 = e1a_s[rows, :]
            x2 = e2a_s[rows, :]
            w = aw_ref[...]
            u = au_ref[...]
            v1 = jnp.tanh(jnp.dot(x1, w, preferred_element_type=F32))
            v2 = jnp.tanh(jnp.dot(x2, w, preferred_element_type=F32))
            s1 = jnp.sum(v1 * u, axis=1, keepdims=True) + 1e-6
            s2 = jnp.sum(v2 * u, axis=1, keepdims=True) + 1e-6
            mx = jnp.maximum(s1, s2)
            q1 = jnp.exp(s1 - mx)
            q2 = jnp.exp(s2 - mx)
            den = q1 + q2
            a1 = q1 / den
            a2 = q2 / den
            alpha_o[...] = jnp.concatenate([a1, a2], axis=1)
            comb = a1 * x1 + a2 * x2
            comb_o[...] = comb
            cb = comb.astype(BF16)
            c1_s[rows, :] = lax.dot_general(
                cb, dwa_ref[...].astype(BF16), _NN,
                preferred_element_type=F32).astype(BF16)
            c2_s[rows, :] = lax.dot_general(
                cb, dwb_ref[...].astype(BF16), _NN,
                preferred_element_type=F32).astype(BF16)

        @pl.when(ph == 5)
        def _h():
            for p_s, c_s, db_ref, h_s in (
                    (p1_s, c1_s, dba_ref, h1_s),
                    (p2_s, c2_s, dbb_ref, h2_s)):
                r = lax.dot_general(p_s[m], c_s[...], _TN,
                                    preferred_element_type=F32)
                h_s[rows, :] = jax.nn.relu(r + db_ref[...]).astype(BF16)

        @pl.when(ph == 6)
        def _g():
            for p_s, h_s, g_o in (
                    (p1_s, h1_s, g1_o),
                    (p2_s, h2_s, g2_o)):
                g_o[...] = lax.dot_general(
                    p_s[m], h_s[...], _TN,
                    preferred_element_type=F32).astype(BF16)

    small = [
        (w2a, (HID, OUT)), (b2a.reshape(1, OUT), (1, OUT)),
        (wd1, (OUT, OUT)), (bd1.reshape(1, 1), (1, 1)),
        (w2b, (HID, OUT)), (b2b.reshape(1, OUT), (1, OUT)),
        (wd2, (OUT, OUT)), (bd2.reshape(1, 1), (1, 1)),
    ]
    small2 = [
        (aw, (OUT, OUT)), (au, (1, OUT)),
        (dwa, (OUT, HID)), (dba.reshape(1, HID), (1, HID)),
        (dwb, (OUT, HID)), (dbb.reshape(1, HID), (1, HID)),
        (b1a.reshape(1, HID), (1, HID)), (b1b.reshape(1, HID), (1, HID)),
    ]
    in_specs = (
        [pl.BlockSpec((N, bm), p_idx)] * 2
        + [pl.BlockSpec((N, HID), const)] * 4
        + [pl.BlockSpec(shp, const) for _, shp in small]
        + [pl.BlockSpec((bm, N), gate(4))] * 2
        + [pl.BlockSpec(shp, const) for _, shp in small2]
    )
    args = ([adj1, adj2, y1a, y1b, y2a, y2b]
            + [a for a, _ in small] + [mask1, mask2]
            + [a for a, _ in small2])
    out_specs = [
        pl.BlockSpec((bm, OUT), gate(3)),
        pl.BlockSpec((bm, OUT), gate(3)),
        pl.BlockSpec((bm, 2), gate(4)),
        pl.BlockSpec((bm, 2), gate(4)),
        pl.BlockSpec((bm, 2), gate(4)),
        pl.BlockSpec((bm, 2), gate(4)),
        pl.BlockSpec((bm, 2), gate(4)),
        pl.BlockSpec((bm, OUT), gate(4)),
        pl.BlockSpec((bm, HID), gate(6)),
        pl.BlockSpec((bm, HID), gate(6)),
    ]
    out_shape = [
        jax.ShapeDtypeStruct((N, OUT), F32),
        jax.ShapeDtypeStruct((N, OUT), F32),
        jax.ShapeDtypeStruct((N, 2), F32),
        jax.ShapeDtypeStruct((N, 2), F32),
        jax.ShapeDtypeStruct((N, 2), F32),
        jax.ShapeDtypeStruct((N, 2), F32),
        jax.ShapeDtypeStruct((N, 2), F32),
        jax.ShapeDtypeStruct((N, OUT), F32),
        jax.ShapeDtypeStruct((N, HID), BF16),
        jax.ShapeDtypeStruct((N, HID), BF16),
    ]
    scratch_shapes = (
        [pltpu.VMEM((nb, N, bm), BF16)] * 2
        + [pltpu.VMEM((N, HID), BF16)] * 4
        + [pltpu.VMEM((N, OUT), F32),
           pltpu.VMEM((N, OUT), BF16),
           pltpu.VMEM((N, OUT), F32),
           pltpu.VMEM((N, OUT), BF16)]
        + [pltpu.VMEM((1, N), F32), pltpu.VMEM((N, 1), F32)] * 2
    )
    return pl.pallas_call(
        body,
        grid=(7, nb),
        in_specs=in_specs,
        out_specs=out_specs,
        out_shape=out_shape,
        scratch_shapes=scratch_shapes,
    )(*args)


def _ptmm2(ps, yss, biases=None, act=None, out_dtype=F32, bm=512,
           pre_ws=None):
    ng = len(ps)
    counts = [len(ys) for ys in yss]
    kcs = [ys[0].shape[1] for ys in yss]
    if pre_ws is not None:
        ncs = [w.shape[1] for w in pre_ws]
    else:
        ncs = kcs

    def body(*refs):
        refs = list(refs)
        p_refs = [refs.pop(0) for _ in range(ng)]
        y_refs = [[refs.pop(0) for _ in range(counts[g])] for g in range(ng)]
        w_refs = ([refs.pop(0) for _ in range(ng)]
                  if pre_ws is not None else [None] * ng)
        b_refs = ([refs.pop(0) for _ in range(ng)]
                  if biases is not None else [None] * ng)
        for g in range(ng):
            s = p_refs[g][...]
            if pre_ws is not None:
                w = w_refs[g][...].astype(BF16)
            for i in range(counts[g]):
                y = y_refs[g][i][...]
                if pre_ws is not None:
                    y = lax.dot_general(y, w, _NN,
                                        preferred_element_type=F32)
                    y = y.astype(BF16)
                r = lax.dot_general(s, y, _TN,
                                    preferred_element_type=F32)
                if biases is not None:
                    r = r + b_refs[g][...]
                if act is not None:
                    r = act(r)
                refs.pop(0)[...] = r.astype(out_dtype)

    in_specs = [pl.BlockSpec((N, bm), lambda m: (0, m))] * ng
    args = list(ps)
    for g in range(ng):
        in_specs += [pl.BlockSpec((N, kcs[g]), lambda m: (0, 0))] * counts[g]
        args += list(yss[g])
    if pre_ws is not None:
        for g in range(ng):
            in_specs.append(
                pl.BlockSpec((kcs[g], ncs[g]), lambda m: (0, 0)))
            args.append(pre_ws[g])
    if biases is not None:
        for g in range(ng):
            in_specs.append(pl.BlockSpec((1, ncs[g]), lambda m: (0, 0)))
            args.append(biases[g].reshape(1, ncs[g]))
    out_specs = []
    out_shape = []
    for g in range(ng):
        out_specs += [pl.BlockSpec((bm, ncs[g]), lambda m: (m, 0))] * counts[g]
        out_shape += [jax.ShapeDtypeStruct((N, ncs[g]), out_dtype)] * counts[g]
    flat = pl.pallas_call(
        body,
        grid=(N // bm,),
        in_specs=in_specs,
        out_specs=out_specs,
        out_shape=out_shape,
    )(*args)
    outs = []
    k = 0
    for g in range(ng):
        outs.append(list(flat[k:k + counts[g]]))
        k += counts[g]
    return outs


def _mm(avs, bmat, bias=None, act=None, out_dtype=F32, bm=512, bn=512):
    nd = len(avs)
    m_, k_ = avs[0].shape
    nc = bmat.shape[1]
    bn = min(bn, -(-nc // 128) * 128)
    grid = (m_ // bm, -(-nc // bn))
    k_main = (k_ // 128) * 128
    ragged = k_main != k_
    ktail_blk = k_main // 128

    def body(*refs):
        refs = list(refs)
        a_refs = [refs.pop(0) for _ in range(nd)]
        if ragged:
            at_refs = [refs.pop(0) for _ in range(nd)]
        b_ref = refs.pop(0)
        if ragged:
            bt_ref = refs.pop(0)
        if bias is not None:
            bias_ref = refs.pop(0)
        o_refs = refs
        bmain = b_ref[...].astype(BF16)
        if ragged:
            kid = lax.broadcasted_iota(jnp.int32, (128, bn), 0)
            btail = jnp.where(kid < (k_ - k_main), bt_ref[...], 0.0)
            btail = btail.astype(BF16)
        for i in range(nd):
            r = lax.dot_general(a_refs[i][...].astype(BF16), bmain, _NN,
                                preferred_element_type=F32)
            if ragged:
                kida = lax.broadcasted_iota(jnp.int32, (bm, 128), 1)
                atail = jnp.where(kida < (k_ - k_main), at_refs[i][...], 0.0)
                r = r + lax.dot_general(atail.astype(BF16), btail, _NN,
                                        preferred_element_type=F32)
            if bias is not None:
                r = r + bias_ref[...]
            if act is not None:
                r = act(r)
            o_refs[i][...] = r.astype(out_dtype)

    in_specs = [pl.BlockSpec((bm, k_main), lambda m, n: (m, 0))] * nd
    args = list(avs)
    if ragged:
        in_specs += [pl.BlockSpec((bm, 128),
                                  lambda m, n: (m, ktail_blk))] * nd
        args += list(avs)
    in_specs.append(pl.BlockSpec((k_main, bn), lambda m, n: (0, n)))
    args.append(bmat)
    if ragged:
        in_specs.append(pl.BlockSpec((128, bn), lambda m, n: (ktail_blk, n)))
        args.append(bmat)
    if bias is not None:
        in_specs.append(pl.BlockSpec((1, bn), lambda m, n: (0, n)))
        args.append(bias)
    return pl.pallas_call(
        body,
        grid=grid,
        in_specs=in_specs,
        out_specs=[pl.BlockSpec((bm, bn), lambda m, n: (m, n))] * nd,
        out_shape=[jax.ShapeDtypeStruct((m_, nc), out_dtype)] * nd,
        compiler_params=_PARAMS,
    )(*args)


def _readout_disc_attn(groups, aw, u_row, dec_w1s):
    bmr = 512
    nh = dec_w1s[0].shape[1]

    def one(mk, ef, eaf, er, ear, w, bdv, ret_ref, reta_ref):
        rs = jnp.sum(mk, axis=1, keepdims=True)
        mb = mk.astype(BF16)
        vs1 = lax.dot_general(mb, ef.astype(BF16), _NN,
                              preferred_element_type=F32)
        vs2 = lax.dot_general(mb, eaf.astype(BF16), _NN,
                              preferred_element_type=F32)
        gp = vs1 / rs
        gap = vs2 / rs
        n1 = jnp.maximum(jnp.sqrt(jnp.sum(gp * gp, axis=1, keepdims=True)),
                         1e-12)
        n2 = jnp.maximum(jnp.sqrt(jnp.sum(gap * gap, axis=1, keepdims=True)),
                         1e-12)
        g = jax.nn.sigmoid(gp / n1)
        ga = jax.nn.sigmoid(gap / n2)
        u = jnp.dot(er, w, preferred_element_type=F32)
        ua = jnp.dot(ear, w, preferred_element_type=F32)
        ret_ref[...] = jnp.concatenate([
            jnp.sum(u * g, axis=1, keepdims=True) + bdv,
            jnp.sum(ua * g, axis=1, keepdims=True) + bdv], axis=1)
        reta_ref[...] = jnp.concatenate([
            jnp.sum(ua * ga, axis=1, keepdims=True) + bdv,
            jnp.sum(u * ga, axis=1, keepdims=True) + bdv], axis=1)

    def body(m1, e1f, e1af, e1r, e1ar, w1, b1,
             m2, e2f, e2af, e2r, e2ar, w2, b2,
             aw_ref, u_ref, dw1_ref, dw2_ref,
             ret1, reta1, ret2, reta2, alpha_ref, comb_ref, c1_ref, c2_ref):
        one(m1[...], e1f[...], e1af[...], e1r[...], e1ar[...], w1[...],
            b1[0, 0], ret1, reta1)
        one(m2[...], e2f[...], e2af[...], e2r[...], e2ar[...], w2[...],
            b2[0, 0], ret2, reta2)
        x1 = e1r[...]
        x2 = e2r[...]
        w = aw_ref[...]
        u = u_ref[...]
        v1 = jnp.tanh(jnp.dot(x1, w, preferred_element_type=F32))
        v2 = jnp.tanh(jnp.dot(x2, w, preferred_element_type=F32))
        s1 = jnp.sum(v1 * u, axis=1, keepdims=True) + 1e-6
        s2 = jnp.sum(v2 * u, axis=1, keepdims=True) + 1e-6
        mx = jnp.maximum(s1, s2)
        p1 = jnp.exp(s1 - mx)
        p2 = jnp.exp(s2 - mx)
        den = p1 + p2
        a1 = p1 / den
        a2 = p2 / den
        alpha_ref[...] = jnp.concatenate([a1, a2], axis=1)
        comb = a1 * x1 + a2 * x2
        comb_ref[...] = comb
        cb = comb.astype(BF16)
        c1_ref[...] = lax.dot_general(
            cb, dw1_ref[...].astype(BF16), _NN,
            preferred_element_type=F32).astype(BF16)
        c2_ref[...] = lax.dot_general(
            cb, dw2_ref[...].astype(BF16), _NN,
            preferred_element_type=F32).astype(BF16)

    specs_one = [
        pl.BlockSpec((bmr, N), lambda m: (m, 0)),
        pl.BlockSpec((N, OUT), lambda m: (0, 0)),
        pl.BlockSpec((N, OUT), lambda m: (0, 0)),
        pl.BlockSpec((bmr, OUT), lambda m: (m, 0)),
        pl.BlockSpec((bmr, OUT), lambda m: (m, 0)),
        pl.BlockSpec((OUT, OUT), lambda m: (0, 0)),
        pl.BlockSpec((1, 1), lambda m: (0, 0)),
    ]
    args = []
    for mask, ec, es, wd, bd in groups:
        args += [mask, ec, es, ec, es, wd, bd.reshape(1, 1)]
    args += [aw, u_row, dec_w1s[0], dec_w1s[1]]
    return pl.pallas_call(
        body,
        grid=(N // bmr,),
        in_specs=specs_one + specs_one + [
            pl.BlockSpec((OUT, OUT), lambda m: (0, 0)),
            pl.BlockSpec((1, OUT), lambda m: (0, 0)),
            pl.BlockSpec((OUT, nh), lambda m: (0, 0)),
            pl.BlockSpec((OUT, nh), lambda m: (0, 0)),
        ],
        out_specs=[pl.BlockSpec((bmr, 2), lambda m: (m, 0))] * 4 + [
            pl.BlockSpec((bmr, 2), lambda m: (m, 0)),
            pl.BlockSpec((bmr, OUT), lambda m: (m, 0)),
            pl.BlockSpec((bmr, nh), lambda m: (m, 0)),
            pl.BlockSpec((bmr, nh), lambda m: (m, 0)),
        ],
        out_shape=[jax.ShapeDtypeStruct((N, 2), F32)] * 4 + [
            jax.ShapeDtypeStruct((N, 2), F32),
            jax.ShapeDtypeStruct((N, OUT), F32),
            jax.ShapeDtypeStruct((N, nh), BF16),
            jax.ShapeDtypeStruct((N, nh), BF16),
        ],
    )(*args)


def _mm_entry(f1c, f1s, b1mat, f2c, f2s, b2mat, bm=512):
    k1 = b1mat.shape[0]
    nc = b1mat.shape[1]
    k1m = (k1 // 128) * 128
    kt = k1m // 128
    k2 = b2mat.shape[0]

    def body(a1_ref, a1t_ref, a2_ref, a2t_ref, b1_ref, b1t_ref,
             a3_ref, a4_ref, b2_ref, o1_ref, o2_ref, o3_ref, o4_ref):
        bmain = b1_ref[...].astype(BF16)
        kid = lax.broadcasted_iota(jnp.int32, (128, nc), 0)
        btail = jnp.where(kid < (k1 - k1m), b1t_ref[...], 0.0).astype(BF16)
        for a_ref, at_ref, o_ref in ((a1_ref, a1t_ref, o1_ref),
                                     (a2_ref, a2t_ref, o2_ref)):
            r = lax.dot_general(a_ref[...].astype(BF16), bmain, _NN,
                                preferred_element_type=F32)
            kida = lax.broadcasted_iota(jnp.int32, (bm, 128), 1)
            atail = jnp.where(kida < (k1 - k1m), at_ref[...], 0.0)
            r = r + lax.dot_general(atail.astype(BF16), btail, _NN,
                                    preferred_element_type=F32)
            o_ref[...] = r.astype(BF16)
        b2v = b2_ref[...].astype(BF16)
        for a_ref, o_ref in ((a3_ref, o3_ref), (a4_ref, o4_ref)):
            o_ref[...] = lax.dot_general(
                a_ref[...].astype(BF16), b2v, _NN,
                preferred_element_type=F32).astype(BF16)

    in_specs = [
        pl.BlockSpec((bm, k1m), lambda m: (m, 0)),
        pl.BlockSpec((bm, 128), lambda m: (m, kt)),
        pl.BlockSpec((bm, k1m), lambda m: (m, 0)),
        pl.BlockSpec((bm, 128), lambda m: (m, kt)),
        pl.BlockSpec((k1m, nc), lambda m: (0, 0)),
        pl.BlockSpec((128, nc), lambda m: (kt, 0)),
        pl.BlockSpec((bm, k2), lambda m: (m, 0)),
        pl.BlockSpec((bm, k2), lambda m: (m, 0)),
        pl.BlockSpec((k2, nc), lambda m: (0, 0)),
    ]
    return pl.pallas_call(
        body,
        grid=(N // bm,),
        in_specs=in_specs,
        out_specs=[pl.BlockSpec((bm, nc), lambda m: (m, 0))] * 4,
        out_shape=[jax.ShapeDtypeStruct((N, nc), BF16)] * 4,
    )(f1c, f1c, f1s, f1s, b1mat, b1mat, f2c, f2s, b2mat)


def _mm_pair(a1, b1, bias1, a2, b2, bias2, bm=512):
    n1 = b1.shape[1]
    n2 = b2.shape[1]
    k = b1.shape[0]

    def body(a1_ref, b1_ref, s1_ref, a2_ref, b2_ref, s2_ref,
             o1_ref, o2_ref):
        for a_ref, b_ref, s_ref, o_ref in (
                (a1_ref, b1_ref, s1_ref, o1_ref),
                (a2_ref, b2_ref, s2_ref, o2_ref)):
            r = lax.dot_general(a_ref[...].astype(BF16),
                                b_ref[...].astype(BF16), _NN,
                                preferred_element_type=F32)
            o_ref[...] = r + s_ref[...]

    def one(nc):
        return [
            pl.BlockSpec((bm, k), lambda m: (m, 0)),
            pl.BlockSpec((k, nc), lambda m: (0, 0)),
            pl.BlockSpec((1, nc), lambda m: (0, 0)),
        ]

    return pl.pallas_call(
        body,
        grid=(N // bm,),
        in_specs=one(n1) + one(n2),
        out_specs=[pl.BlockSpec((bm, n1), lambda m: (m, 0)),
                   pl.BlockSpec((bm, n2), lambda m: (m, 0))],
        out_shape=[jax.ShapeDtypeStruct((N, n1), F32),
                   jax.ShapeDtypeStruct((N, n2), F32)],
    )(a1, b1, bias1.reshape(1, n1), a2, b2, bias2.reshape(1, n2))


def kernel(omics1_feat_shuffle, omics2_feat_shuffle, omics1_feat, omics2_feat,
           omics1_adj, omics2_adj, omics1_graph_neigh, omics2_graph_neigh,
           o1_enc_W1, o1_enc_b1, o1_enc_W2, o1_enc_b2, o1_disc_W, o1_disc_b,
           o2_enc_W1, o2_enc_b1, o2_enc_W2, o2_enc_b2, o2_disc_W, o2_disc_b,
           o1_dec_W1, o1_dec_b1, o1_dec_W2, o1_dec_b2,
           o2_dec_W1, o2_dec_b1, o2_dec_W2, o2_dec_b2, att_w, att_u):
    y1a, y1b, y2a, y2b = _mm_entry(
        omics1_feat, omics1_feat_shuffle, o1_enc_W1,
        omics2_feat, omics2_feat_shuffle, o2_enc_W1)

    (e1a, e2a, o1_ret, o1_ret_a, o2_ret, o2_ret_a,
     alpha, comb, g1, g2) = _core(
        omics1_adj, omics2_adj, y1a, y1b, y2a, y2b,
        o1_enc_W2, o1_enc_b2, o1_disc_W, o1_disc_b,
        o2_enc_W2, o2_enc_b2, o2_disc_W, o2_disc_b,
        omics1_graph_neigh, omics2_graph_neigh,
        att_w, att_u.reshape(1, OUT),
        o1_dec_W1, o1_dec_b1, o2_dec_W1, o2_dec_b1,
        o1_enc_b1, o2_enc_b1)

    o1_rec, o2_rec = _mm_pair(g1, o1_dec_W2, o1_dec_b2,
                              g2, o2_dec_W2, o2_dec_b2)

    return (e1a, o1_rec, o1_ret, o1_ret_a,
            e2a, o2_rec, o2_ret, o2_ret_a, comb, alpha)


# speedup vs baseline: 1.3522x; 1.0060x over previous
import jax
import jax.numpy as jnp
from jax import lax
from jax.experimental import pallas as pl
from jax.experimental.pallas import tpu as pltpu

N = 2048
HID = 256
OUT = 128
F32 = jnp.float32
BF16 = jnp.bfloat16

_TN = (((0,), (0,)), ((), ()))
_NN = (((1,), (0,)), ((), ()))
_PARAMS = pltpu.CompilerParams(
    dimension_semantics=("parallel", "parallel"))


def _core(adj1, adj2, y1a, y1b, y2a, y2b,
          w2a, b2a, wd1, bd1, w2b, b2b, wd2, bd2,
          mask1, mask2, aw, au, dwa, dba, dwb, dbb,
          b1a, b1b):
    bm = 256
    nb = N // bm

    def gate(phx):
        return lambda ph, m: (jnp.where(ph < phx, 0,
                                        jnp.where(ph == phx, m, nb - 1)), 0)

    def const(ph, m):
        return (0, 0)

    p_idx = lambda ph, m: (0, jnp.where(ph == 0, m, nb - 1))

    def readout_one(mk, ef, eaf_bf, er, ear_bf, w, bdv, ret_ref, reta_ref):
        rs = jnp.sum(mk, axis=1, keepdims=True)
        mb = mk.astype(BF16)
        vs1 = lax.dot_general(mb, ef.astype(BF16), _NN,
                              preferred_element_type=F32)
        vs2 = lax.dot_general(mb, eaf_bf, _NN,
                              preferred_element_type=F32)
        ear = ear_bf.astype(F32)
        gp = vs1 / rs
        gap = vs2 / rs
        n1 = jnp.maximum(jnp.sqrt(jnp.sum(gp * gp, axis=1, keepdims=True)),
                         1e-12)
        n2 = jnp.maximum(jnp.sqrt(jnp.sum(gap * gap, axis=1, keepdims=True)),
                         1e-12)
        g = jax.nn.sigmoid(gp / n1)
        ga = jax.nn.sigmoid(gap / n2)
        u = jnp.dot(er, w, preferred_element_type=F32)
        ua = jnp.dot(ear, w, preferred_element_type=F32)
        ret_ref[...] = jnp.concatenate([
            jnp.sum(u * g, axis=1, keepdims=True) + bdv,
            jnp.sum(ua * g, axis=1, keepdims=True) + bdv], axis=1)
        reta_ref[...] = jnp.concatenate([
            jnp.sum(ua * ga, axis=1, keepdims=True) + bdv,
            jnp.sum(u * ga, axis=1, keepdims=True) + bdv], axis=1)

    def body(a1_ref, a2_ref, y1a_ref, y1b_ref, y2a_ref, y2b_ref,
             w2a_ref, b2a_ref, wd1_ref, bd1_ref,
             w2b_ref, b2b_ref, wd2_ref, bd2_ref,
             m1_ref, m2_ref, aw_ref, au_ref,
             dwa_ref, dba_ref, dwb_ref, dbb_ref, b1a_ref, b1b_ref,
             e1_o, e2_o, ret1_o, reta1_o, ret2_o, reta2_o,
             alpha_o, comb_o, g1_o, g2_o,
             p1_s, p2_s,
             z1a_s, z1b_s, z2a_s, z2b_s,
             e1a_s, e1b_s, e2a_s, e2b_s,
             dvr1_s, dvc1_s, dvr2_s, dvc2_s):
        ph = pl.program_id(0)
        m = pl.program_id(1)
        rows = pl.ds(m * bm, bm)
        c1_s, c2_s, h1_s, h2_s = z1a_s, z1b_s, z2a_s, z2b_s

        @pl.when(ph == 0)
        def _prep():
            cols = pl.ds(m * bm, bm)
            for a_ref, p_s, dvr_s, dvc_s in (
                    (a1_ref, p1_s, dvr1_s, dvc1_s),
                    (a2_ref, p2_s, dvr2_s, dvc2_s)):
                t = a_ref[...]
                b = jnp.where(t != 0, 1.0, 0.0)
                rid = lax.broadcasted_iota(jnp.int32, t.shape, 0)
                cid = m * bm + lax.broadcasted_iota(jnp.int32, t.shape, 1)
                d = jnp.where(rid == cid, 1.0, b)
                p_s[m] = d.astype(BF16)
                dvr_s[0:1, cols] = 1.0 / jnp.sqrt(
                    jnp.sum(d, axis=0, keepdims=True))
                deg = lax.dot_general(d, jnp.ones((N, 1), F32), _TN,
                                      preferred_element_type=F32)
                dvc_s[cols, :] = 1.0 / jnp.sqrt(deg)

        @pl.when(ph == 1)
        def _norm():
            cols = pl.ds(m * bm, bm)
            for p_s, dvr_s, dvc_s in (
                    (p1_s, dvr1_s, dvc1_s),
                    (p2_s, dvr2_s, dvc2_s)):
                dr = dvr_s[0:1, cols]
                dc = dvc_s[...]
                p_s[m] = ((dc * p_s[m].astype(F32)) * dr).astype(BF16)

        @pl.when(ph == 2)
        def _l1():
            for p_s, ys, b_ref, zs in (
                    (p1_s, (y1a_ref, y1b_ref), b1a_ref, (z1a_s, z1b_s)),
                    (p2_s, (y2a_ref, y2b_ref), b1b_ref, (z2a_s, z2b_s))):
                s = p_s[m]
                for y_ref, z_ref in zip(ys, zs):
                    r = lax.dot_general(s, y_ref[...], _TN,
                                        preferred_element_type=F32)
                    z_ref[rows, :] = jax.nn.relu(r + b_ref[...]).astype(BF16)

        @pl.when(ph == 3)
        def _l2():
            for p_s, zs, w_ref, b_ref, es, e_out in (
                    (p1_s, (z1a_s, z1b_s), w2a_ref, b2a_ref,
                     (e1a_s, e1b_s), e1_o),
                    (p2_s, (z2a_s, z2b_s), w2b_ref, b2b_ref,
                     (e2a_s, e2b_s), e2_o)):
                s = p_s[m]
                w = w_ref[...].astype(BF16)
                for i, (z_ref, e_ref) in enumerate(zip(zs, es)):
                    q = lax.dot_general(z_ref[...], w, _NN,
                                        preferred_element_type=F32)
                    r = lax.dot_general(s, q.astype(BF16), _TN,
                                        preferred_element_type=F32)
                    r = r + b_ref[...]
                    if i == 0:
                        e_ref[rows, :] = r
                        e_out[...] = r
                    else:
                        e_ref[rows, :] = r.astype(BF16)

        @pl.when(ph == 4)
        def _readout():
            readout_one(m1_ref[...], e1a_s[...], e1b_s[...],
                        e1a_s[rows, :], e1b_s[rows, :], wd1_ref[...],
                        bd1_ref[0, 0], ret1_o, reta1_o)
            readout_one(m2_ref[...], e2a_s[...], e2b_s[...],
                        e2a_s[rows, :], e2b_s[rows, :], wd2_ref[...],
                        bd2_ref[0, 0], ret2_o, reta2_o)
            x1 = e1a_s[rows, :]
            x2 = e2a_s[rows, :]
            w = aw_ref[...]
            u = au_ref[...]
            v1 = jnp.tanh(jnp.dot(x1, w, preferred_element_type=F32))
            v2 = jnp.tanh(jnp.dot(x2, w, preferred_element_type=F32))
            s1 = jnp.sum(v1 * u, axis=1, keepdims=True) + 1e-6
            s2 = jnp.sum(v2 * u, axis=1, keepdims=True) + 1e-6
            mx = jnp.maximum(s1, s2)
            q1 = jnp.exp(s1 - mx)
            q2 = jnp.exp(s2 - mx)
            den = q1 + q2
            a1 = q1 / den
            a2 = q2 / den
            alpha_o[...] = jnp.concatenate([a1, a2], axis=1)
            comb = a1 * x1 + a2 * x2
            comb_o[...] = comb
            cb = comb.astype(BF16)
            c1_s[rows, :] = lax.dot_general(
                cb, dwa_ref[...].astype(BF16), _NN,
                preferred_element_type=F32).astype(BF16)
            c2_s[rows, :] = lax.dot_general(
                cb, dwb_ref[...].astype(BF16), _NN,
                preferred_element_type=F32).astype(BF16)

        @pl.when(ph == 5)
        def _h():
            for p_s, c_s, db_ref, h_s in (
                    (p1_s, c1_s, dba_ref, h1_s),
                    (p2_s, c2_s, dbb_ref, h2_s)):
                r = lax.dot_general(p_s[m], c_s[...], _TN,
                                    preferred_element_type=F32)
                h_s[rows, :] = jax.nn.relu(r + db_ref[...]).astype(BF16)

        @pl.when(ph == 6)
        def _g():
            for p_s, h_s, g_o in (
                    (p1_s, h1_s, g1_o),
                    (p2_s, h2_s, g2_o)):
                g_o[...] = lax.dot_general(
                    p_s[m], h_s[...], _TN,
                    preferred_element_type=F32).astype(BF16)

    small = [
        (w2a, (HID, OUT)), (b2a.reshape(1, OUT), (1, OUT)),
        (wd1, (OUT, OUT)), (bd1.reshape(1, 1), (1, 1)),
        (w2b, (HID, OUT)), (b2b.reshape(1, OUT), (1, OUT)),
        (wd2, (OUT, OUT)), (bd2.reshape(1, 1), (1, 1)),
    ]
    small2 = [
        (aw, (OUT, OUT)), (au, (1, OUT)),
        (dwa, (OUT, HID)), (dba.reshape(1, HID), (1, HID)),
        (dwb, (OUT, HID)), (dbb.reshape(1, HID), (1, HID)),
        (b1a.reshape(1, HID), (1, HID)), (b1b.reshape(1, HID), (1, HID)),
    ]
    in_specs = (
        [pl.BlockSpec((N, bm), p_idx)] * 2
        + [pl.BlockSpec((N, HID), const)] * 4
        + [pl.BlockSpec(shp, const) for _, shp in small]
        + [pl.BlockSpec((bm, N), gate(4))] * 2
        + [pl.BlockSpec(shp, const) for _, shp in small2]
    )
    args = ([adj1, adj2, y1a, y1b, y2a, y2b]
            + [a for a, _ in small] + [mask1, mask2]
            + [a for a, _ in small2])
    out_specs = [
        pl.BlockSpec((bm, OUT), gate(3)),
        pl.BlockSpec((bm, OUT), gate(3)),
        pl.BlockSpec((bm, 2), gate(4)),
        pl.BlockSpec((bm, 2), gate(4)),
        pl.BlockSpec((bm, 2), gate(4)),
        pl.BlockSpec((bm, 2), gate(4)),
        pl.BlockSpec((bm, 2), gate(4)),
        pl.BlockSpec((bm, OUT), gate(4)),
        pl.BlockSpec((bm, HID), gate(6)),
        pl.BlockSpec((bm, HID), gate(6)),
    ]
    out_shape = [
        jax.ShapeDtypeStruct((N, OUT), F32),
        jax.ShapeDtypeStruct((N, OUT), F32),
        jax.ShapeDtypeStruct((N, 2), F32),
        jax.ShapeDtypeStruct((N, 2), F32),
        jax.ShapeDtypeStruct((N, 2), F32),
        jax.ShapeDtypeStruct((N, 2), F32),
        jax.ShapeDtypeStruct((N, 2), F32),
        jax.ShapeDtypeStruct((N, OUT), F32),
        jax.ShapeDtypeStruct((N, HID), BF16),
        jax.ShapeDtypeStruct((N, HID), BF16),
    ]
    scratch_shapes = (
        [pltpu.VMEM((nb, N, bm), BF16)] * 2
        + [pltpu.VMEM((N, HID), BF16)] * 4
        + [pltpu.VMEM((N, OUT), F32),
           pltpu.VMEM((N, OUT), BF16),
           pltpu.VMEM((N, OUT), F32),
           pltpu.VMEM((N, OUT), BF16)]
        + [pltpu.VMEM((1, N), F32), pltpu.VMEM((N, 1), F32)] * 2
    )
    return pl.pallas_call(
        body,
        grid=(7, nb),
        in_specs=in_specs,
        out_specs=out_specs,
        out_shape=out_shape,
        scratch_shapes=scratch_shapes,
    )(*args)


def _mm_entry(f1c, f1s, b1mat, f2c, f2s, b2mat, bm=512):
    k1 = b1mat.shape[0]
    nc = b1mat.shape[1]
    k1m = (k1 // 128) * 128
    kt = k1m // 128
    k2 = b2mat.shape[0]

    def body(a1_ref, a1t_ref, a2_ref, a2t_ref, b1_ref, b1t_ref,
             a3_ref, a4_ref, b2_ref, o1_ref, o2_ref, o3_ref, o4_ref):
        bmain = b1_ref[...].astype(BF16)
        kid = lax.broadcasted_iota(jnp.int32, (128, nc), 0)
        btail = jnp.where(kid < (k1 - k1m), b1t_ref[...], 0.0).astype(BF16)
        for a_ref, at_ref, o_ref in ((a1_ref, a1t_ref, o1_ref),
                                     (a2_ref, a2t_ref, o2_ref)):
            r = lax.dot_general(a_ref[...].astype(BF16), bmain, _NN,
                                preferred_element_type=F32)
            kida = lax.broadcasted_iota(jnp.int32, (bm, 128), 1)
            atail = jnp.where(kida < (k1 - k1m), at_ref[...], 0.0)
            r = r + lax.dot_general(atail.astype(BF16), btail, _NN,
                                    preferred_element_type=F32)
            o_ref[...] = r.astype(BF16)
        b2v = b2_ref[...].astype(BF16)
        for a_ref, o_ref in ((a3_ref, o3_ref), (a4_ref, o4_ref)):
            o_ref[...] = lax.dot_general(
                a_ref[...].astype(BF16), b2v, _NN,
                preferred_element_type=F32).astype(BF16)

    in_specs = [
        pl.BlockSpec((bm, k1m), lambda m: (m, 0)),
        pl.BlockSpec((bm, 128), lambda m: (m, kt)),
        pl.BlockSpec((bm, k1m), lambda m: (m, 0)),
        pl.BlockSpec((bm, 128), lambda m: (m, kt)),
        pl.BlockSpec((k1m, nc), lambda m: (0, 0)),
        pl.BlockSpec((128, nc), lambda m: (kt, 0)),
        pl.BlockSpec((bm, k2), lambda m: (m, 0)),
        pl.BlockSpec((bm, k2), lambda m: (m, 0)),
        pl.BlockSpec((k2, nc), lambda m: (0, 0)),
    ]
    return pl.pallas_call(
        body,
        grid=(N // bm,),
        in_specs=in_specs,
        out_specs=[pl.BlockSpec((bm, nc), lambda m: (m, 0))] * 4,
        out_shape=[jax.ShapeDtypeStruct((N, nc), BF16)] * 4,
    )(f1c, f1c, f1s, f1s, b1mat, b1mat, f2c, f2s, b2mat)


def _mm_pair(a1, b1, bias1, a2, b2, bias2, bm=512):
    n1 = b1.shape[1]
    n2 = b2.shape[1]
    k = b1.shape[0]

    def body(a1_ref, b1_ref, s1_ref, a2_ref, b2_ref, s2_ref,
             o1_ref, o2_ref):
        for a_ref, b_ref, s_ref, o_ref in (
                (a1_ref, b1_ref, s1_ref, o1_ref),
                (a2_ref, b2_ref, s2_ref, o2_ref)):
            r = lax.dot_general(a_ref[...].astype(BF16),
                                b_ref[...].astype(BF16), _NN,
                                preferred_element_type=F32)
            o_ref[...] = r + s_ref[...]

    def one(nc):
        return [
            pl.BlockSpec((bm, k), lambda m: (m, 0)),
            pl.BlockSpec((k, nc), lambda m: (0, 0)),
            pl.BlockSpec((1, nc), lambda m: (0, 0)),
        ]

    return pl.pallas_call(
        body,
        grid=(N // bm,),
        in_specs=one(n1) + one(n2),
        out_specs=[pl.BlockSpec((bm, n1), lambda m: (m, 0)),
                   pl.BlockSpec((bm, n2), lambda m: (m, 0))],
        out_shape=[jax.ShapeDtypeStruct((N, n1), F32),
                   jax.ShapeDtypeStruct((N, n2), F32)],
    )(a1, b1, bias1.reshape(1, n1), a2, b2, bias2.reshape(1, n2))


def kernel(omics1_feat_shuffle, omics2_feat_shuffle, omics1_feat, omics2_feat,
           omics1_adj, omics2_adj, omics1_graph_neigh, omics2_graph_neigh,
           o1_enc_W1, o1_enc_b1, o1_enc_W2, o1_enc_b2, o1_disc_W, o1_disc_b,
           o2_enc_W1, o2_enc_b1, o2_enc_W2, o2_enc_b2, o2_disc_W, o2_disc_b,
           o1_dec_W1, o1_dec_b1, o1_dec_W2, o1_dec_b2,
           o2_dec_W1, o2_dec_b1, o2_dec_W2, o2_dec_b2, att_w, att_u):
    y1a, y1b, y2a, y2b = _mm_entry(
        omics1_feat, omics1_feat_shuffle, o1_enc_W1,
        omics2_feat, omics2_feat_shuffle, o2_enc_W1)

    (e1a, e2a, o1_ret, o1_ret_a, o2_ret, o2_ret_a,
     alpha, comb, g1, g2) = _core(
        omics1_adj, omics2_adj, y1a, y1b, y2a, y2b,
        o1_enc_W2, o1_enc_b2, o1_disc_W, o1_disc_b,
        o2_enc_W2, o2_enc_b2, o2_disc_W, o2_disc_b,
        omics1_graph_neigh, omics2_graph_neigh,
        att_w, att_u.reshape(1, OUT),
        o1_dec_W1, o1_dec_b1, o2_dec_W1, o2_dec_b1,
        o1_enc_b1, o2_enc_b1)

    o1_rec, o2_rec = _mm_pair(g1, o1_dec_W2, o1_dec_b2,
                              g2, o2_dec_W2, o2_dec_b2)

    return (e1a, o1_rec, o1_ret, o1_ret_a,
            e2a, o2_rec, o2_ret, o2_ret_a, comb, alpha)
